# Initial kernel scaffold; baseline (speedup 1.0000x reference)
import jax
import jax.numpy as jnp
from jax.experimental import pallas as pl


def kernel(node_features, edge_features, edge_index, P_weights, Q_weights, pool_W, pool_b, W_weight):
    raise NotImplementedError("write your pallas kernel here")


# speedup vs baseline: 1.6850x; 1.6850x over previous
import functools

import jax
import jax.numpy as jnp
from jax import lax
from jax.experimental import pallas as pl
from jax.experimental.pallas import tpu as pltpu

N = 256
E = 8192
D = 128
EBLK = 512
NBLK = E // EBLK


def _relu(x):
    return jnp.maximum(x, 0.0)


def _dot(a, b):
    return jax.lax.dot_general(a, b, (((1,), (0,)), ((), ())),
                               preferred_element_type=jnp.float32)


def _dot_t(a, b):
    return jax.lax.dot_general(a, b, (((1,), (1,)), ((), ())),
                               preferred_element_type=jnp.float32)


def _dot_c0(a, b):
    return jax.lax.dot_general(a, b, (((0,), (0,)), ((), ())),
                               preferred_element_type=jnp.float32)


def _k1_body(nf_ref, ef_ref, rvf_ref, src_ref, dst_ref, rp1_ref,
             pw_ref, poolw_ref, poolb_ref, s_out, nodep1_s, sacc_s):
    i = pl.program_id(0)

    @pl.when(i == 0)
    def _init():
        nodep1_s[...] = _dot(nf_ref[...], pw_ref[0:D, :])
        sacc_s[...] = jnp.zeros((N, D), jnp.float32)

    src_row = src_ref[0]
    dst_row = dst_ref[0]
    rp1_row = rp1_ref[0]
    iota_n = lax.broadcasted_iota(jnp.int32, (N, EBLK), 0)
    ohs = (iota_n == src_row).astype(jnp.float32)
    ohd = (iota_n == dst_row).astype(jnp.float32)
    maskb = (rp1_row == 0).astype(jnp.float32)
    ohd_m = ohd * maskb

    ef_b = ef_ref[...]
    rv_b = rvf_ref[...]
    p2 = pw_ref[D:2 * D, :]
    p3 = pw_ref[2 * D:3 * D, :]
    a_f = _dot(ef_b, p2)
    b_f = _dot(ef_b, p3)
    r_f = _dot(rv_b, p3)
    n_s = _dot_c0(ohs, nodep1_s[...])
    n_d = _dot_c0(ohd, nodep1_s[...])

    poolb = poolb_ref[...]
    relub = _relu(poolb)
    h1a = _relu(n_s + a_f + r_f)
    h2a = _relu(_dot_t(h1a, poolw_ref[...]) + poolb)
    h1b = _relu(n_d + b_f)
    h2b = _relu(_dot_t(h1b, poolw_ref[...]) + poolb)
    ca = h2a - relub
    cb = h2b - relub
    sacc_s[...] += _dot(ohs, ca) + _dot(ohd_m, cb)

    @pl.when(i == NBLK - 1)
    def _fin():
        s_out[...] = sacc_s[...]


def _k2_body(nf_ref, ef_ref, src_ref, dst_ref, s_ref, qw_ref, poolb_ref,
             ww_ref, qf_out, wf_out, qf_s, qw2_s, qw3_s):
    i = pl.program_id(0)

    @pl.when(i == 0)
    def _init():
        relub = _relu(poolb_ref[...])
        pf = relub + s_ref[...] * (1.0 / (N - 1))
        qf = _relu(_dot(nf_ref[...], qw_ref[0:D, :]) +
                   _dot(pf, qw_ref[D:2 * D, :]))
        qf_s[...] = qf
        qf_out[...] = qf
        qw2_s[...] = _dot(qf, ww_ref[D:2 * D, :])
        qw3_s[...] = _dot(qf, ww_ref[2 * D:3 * D, :])

    src_row = src_ref[0]
    dst_row = dst_ref[0]
    iota_n = lax.broadcasted_iota(jnp.int32, (N, EBLK), 0)
    ohs = (iota_n == src_row).astype(jnp.float32)
    ohd = (iota_n == dst_row).astype(jnp.float32)
    wf_out[...] = _relu(_dot(ef_ref[...], ww_ref[0:D, :]) +
                        _dot_c0(ohs, qw2_s[...]) +
                        _dot_c0(ohd, qw3_s[...]))


def _pair_pool(nf, ef, revfeat, src3, dst3, rp13, P_weights, pool_W, pool_b):
    return pl.pallas_call(
        _k1_body,
        grid=(NBLK,),
        in_specs=[
            pl.BlockSpec((N, D), lambda i: (0, 0)),
            pl.BlockSpec((EBLK, D), lambda i: (i, 0)),
            pl.BlockSpec((EBLK, D), lambda i: (i, 0)),
            pl.BlockSpec((1, 1, EBLK), lambda i: (i, 0, 0)),
            pl.BlockSpec((1, 1, EBLK), lambda i: (i, 0, 0)),
            pl.BlockSpec((1, 1, EBLK), lambda i: (i, 0, 0)),
            pl.BlockSpec((3 * D, D), lambda i: (0, 0)),
            pl.BlockSpec((D, D), lambda i: (0, 0)),
            pl.BlockSpec((1, D), lambda i: (0, 0)),
        ],
        out_specs=pl.BlockSpec((N, D), lambda i: (0, 0)),
        out_shape=jax.ShapeDtypeStruct((N, D), jnp.float32),
        scratch_shapes=[
            pltpu.VMEM((N, D), jnp.float32),
            pltpu.VMEM((N, D), jnp.float32),
        ],
    )(nf, ef, revfeat, src3, dst3, rp13, P_weights, pool_W, pool_b)


def _qf_wf(nf, ef, src3, dst3, s_sum, Q_weights, pool_b, W_weight):
    return pl.pallas_call(
        _k2_body,
        grid=(NBLK,),
        in_specs=[
            pl.BlockSpec((N, D), lambda i: (0, 0)),
            pl.BlockSpec((EBLK, D), lambda i: (i, 0)),
            pl.BlockSpec((1, 1, EBLK), lambda i: (i, 0, 0)),
            pl.BlockSpec((1, 1, EBLK), lambda i: (i, 0, 0)),
            pl.BlockSpec((N, D), lambda i: (0, 0)),
            pl.BlockSpec((2 * D, D), lambda i: (0, 0)),
            pl.BlockSpec((1, D), lambda i: (0, 0)),
            pl.BlockSpec((3 * D, D), lambda i: (0, 0)),
        ],
        out_specs=[
            pl.BlockSpec((N, D), lambda i: (0, 0)),
            pl.BlockSpec((EBLK, D), lambda i: (i, 0)),
        ],
        out_shape=[
            jax.ShapeDtypeStruct((N, D), jnp.float32),
            jax.ShapeDtypeStruct((E, D), jnp.float32),
        ],
        scratch_shapes=[
            pltpu.VMEM((N, D), jnp.float32),
            pltpu.VMEM((N, D), jnp.float32),
            pltpu.VMEM((N, D), jnp.float32),
        ],
    )(nf, ef, src3, dst3, s_sum, Q_weights, pool_b, W_weight)


def kernel(node_features, edge_features, edge_index, P_weights, Q_weights,
           pool_W, pool_b, W_weight):
    src = edge_index[:, 0]
    dst = edge_index[:, 1]
    lin = src * N + dst
    rlin = dst * N + src
    table = jnp.full((N * N,), -1, jnp.int32).at[lin].set(
        jnp.arange(E, dtype=jnp.int32))
    rp1 = table[rlin] + 1
    efpad = jnp.concatenate(
        [jnp.zeros((1, D), jnp.float32), edge_features], axis=0)
    revfeat = jnp.take(efpad, rp1, axis=0)

    src3 = src.reshape(NBLK, 1, EBLK)
    dst3 = dst.reshape(NBLK, 1, EBLK)
    rp13 = rp1.reshape(NBLK, 1, EBLK)
    pool_b2 = pool_b.reshape(1, D)

    s_sum = _pair_pool(node_features, edge_features, revfeat, src3, dst3,
                       rp13, P_weights, pool_W, pool_b2)
    qf, wf = _qf_wf(node_features, edge_features, src3, dst3, s_sum,
                    Q_weights, pool_b2, W_weight)
    return (qf, wf)


# speedup vs baseline: 1.8551x; 1.1009x over previous
import functools

import jax
import jax.numpy as jnp
from jax import lax
from jax.experimental import pallas as pl
from jax.experimental.pallas import tpu as pltpu
from jax.experimental.pallas import tpu_sc as plsc

N = 256
E = 8192
D = 128
EBLK = 512
NBLK = E // EBLK
NWORK = 32
EPW = E // NWORK


def _relu(x):
    return jnp.maximum(x, 0.0)


def _dot(a, b):
    return jax.lax.dot_general(a, b, (((1,), (0,)), ((), ())),
                               preferred_element_type=jnp.float32)


def _dot_t(a, b):
    return jax.lax.dot_general(a, b, (((1,), (1,)), ((), ())),
                               preferred_element_type=jnp.float32)


def _dot_c0(a, b):
    return jax.lax.dot_general(a, b, (((0,), (0,)), ((), ())),
                               preferred_element_type=jnp.float32)


def _k1_body(nf_ref, ef_ref, rvf_ref, src_ref, dst_ref, rp1_ref,
             pw_ref, poolw_ref, poolb_ref, s_out, nodep1_s, sacc_s):
    i = pl.program_id(0)

    @pl.when(i == 0)
    def _init():
        nodep1_s[...] = _dot(nf_ref[...], pw_ref[0:D, :])
        sacc_s[...] = jnp.zeros((N, D), jnp.float32)

    src_row = src_ref[0]
    dst_row = dst_ref[0]
    rp1_row = rp1_ref[0]
    iota_n = lax.broadcasted_iota(jnp.int32, (N, EBLK), 0)
    ohs = (iota_n == src_row).astype(jnp.float32)
    ohd = (iota_n == dst_row).astype(jnp.float32)
    maskb = (rp1_row == 0).astype(jnp.float32)
    ohd_m = ohd * maskb

    ef_b = ef_ref[...]
    rv_b = rvf_ref[...]
    p2 = pw_ref[D:2 * D, :]
    p3 = pw_ref[2 * D:3 * D, :]
    a_f = _dot(ef_b, p2)
    b_f = _dot(ef_b, p3)
    r_f = _dot(rv_b, p3)
    n_s = _dot_c0(ohs, nodep1_s[...])
    n_d = _dot_c0(ohd, nodep1_s[...])

    poolb = poolb_ref[...]
    relub = _relu(poolb)
    h1a = _relu(n_s + a_f + r_f)
    h2a = _relu(_dot_t(h1a, poolw_ref[...]) + poolb)
    h1b = _relu(n_d + b_f)
    h2b = _relu(_dot_t(h1b, poolw_ref[...]) + poolb)
    ca = h2a - relub
    cb = h2b - relub
    sacc_s[...] += _dot(ohs, ca) + _dot(ohd_m, cb)

    @pl.when(i == NBLK - 1)
    def _fin():
        s_out[...] = sacc_s[...]


def _k2_body(nf_ref, ef_ref, src_ref, dst_ref, s_ref, qw_ref, poolb_ref,
             ww_ref, qf_out, wf_out, qf_s, qw2_s, qw3_s):
    i = pl.program_id(0)

    @pl.when(i == 0)
    def _init():
        relub = _relu(poolb_ref[...])
        pf = relub + s_ref[...] * (1.0 / (N - 1))
        qf = _relu(_dot(nf_ref[...], qw_ref[0:D, :]) +
                   _dot(pf, qw_ref[D:2 * D, :]))
        qf_s[...] = qf
        qf_out[...] = qf
        qw2_s[...] = _dot(qf, ww_ref[D:2 * D, :])
        qw3_s[...] = _dot(qf, ww_ref[2 * D:3 * D, :])

    src_row = src_ref[0]
    dst_row = dst_ref[0]
    iota_n = lax.broadcasted_iota(jnp.int32, (N, EBLK), 0)
    ohs = (iota_n == src_row).astype(jnp.float32)
    ohd = (iota_n == dst_row).astype(jnp.float32)
    wf_out[...] = _relu(_dot(ef_ref[...], ww_ref[0:D, :]) +
                        _dot_c0(ohs, qw2_s[...]) +
                        _dot_c0(ohd, qw3_s[...]))


def _pair_pool(nf, ef, revfeat, src3, dst3, rp13, P_weights, pool_W, pool_b):
    return pl.pallas_call(
        _k1_body,
        grid=(NBLK,),
        in_specs=[
            pl.BlockSpec((N, D), lambda i: (0, 0)),
            pl.BlockSpec((EBLK, D), lambda i: (i, 0)),
            pl.BlockSpec((EBLK, D), lambda i: (i, 0)),
            pl.BlockSpec((1, 1, EBLK), lambda i: (i, 0, 0)),
            pl.BlockSpec((1, 1, EBLK), lambda i: (i, 0, 0)),
            pl.BlockSpec((1, 1, EBLK), lambda i: (i, 0, 0)),
            pl.BlockSpec((3 * D, D), lambda i: (0, 0)),
            pl.BlockSpec((D, D), lambda i: (0, 0)),
            pl.BlockSpec((1, D), lambda i: (0, 0)),
        ],
        out_specs=pl.BlockSpec((N, D), lambda i: (0, 0)),
        out_shape=jax.ShapeDtypeStruct((N, D), jnp.float32),
        scratch_shapes=[
            pltpu.VMEM((N, D), jnp.float32),
            pltpu.VMEM((N, D), jnp.float32),
        ],
    )(nf, ef, revfeat, src3, dst3, rp13, P_weights, pool_W, pool_b)


def _qf_wf(nf, ef, src3, dst3, s_sum, Q_weights, pool_b, W_weight):
    return pl.pallas_call(
        _k2_body,
        grid=(NBLK,),
        in_specs=[
            pl.BlockSpec((N, D), lambda i: (0, 0)),
            pl.BlockSpec((EBLK, D), lambda i: (i, 0)),
            pl.BlockSpec((1, 1, EBLK), lambda i: (i, 0, 0)),
            pl.BlockSpec((1, 1, EBLK), lambda i: (i, 0, 0)),
            pl.BlockSpec((N, D), lambda i: (0, 0)),
            pl.BlockSpec((2 * D, D), lambda i: (0, 0)),
            pl.BlockSpec((1, D), lambda i: (0, 0)),
            pl.BlockSpec((3 * D, D), lambda i: (0, 0)),
        ],
        out_specs=[
            pl.BlockSpec((N, D), lambda i: (0, 0)),
            pl.BlockSpec((EBLK, D), lambda i: (i, 0)),
        ],
        out_shape=[
            jax.ShapeDtypeStruct((N, D), jnp.float32),
            jax.ShapeDtypeStruct((E, D), jnp.float32),
        ],
        scratch_shapes=[
            pltpu.VMEM((N, D), jnp.float32),
            pltpu.VMEM((N, D), jnp.float32),
            pltpu.VMEM((N, D), jnp.float32),
        ],
    )(nf, ef, src3, dst3, s_sum, Q_weights, pool_b, W_weight)


def _sc_rev_body(src_hbm, dst_hbm, efpad_hbm, rp1_hbm, revfeat_hbm,
                 src_v, dst_v, lin_v, table_v, idx_v, rows_v, sem):
    wid = lax.axis_index("s") * 2 + lax.axis_index("c")
    pltpu.sync_copy(src_hbm, src_v)
    pltpu.sync_copy(dst_hbm, dst_v)
    iota = lax.iota(jnp.int32, 16)

    def scatter_body(i, carry):
        sv = src_v[pl.ds(i * 16, 16)]
        dv = dst_v[pl.ds(i * 16, 16)]
        linv = sv * N + dv
        lin_v[pl.ds(i * 16, 16)] = linv
        plsc.store_scatter(table_v, [linv], iota + i * 16)
        return carry

    lax.fori_loop(0, E // 16, scatter_body, 0)

    base = wid * EPW

    def resolve_body(i, carry):
        off = base + i * 16
        sv = src_v[pl.ds(off, 16)]
        dv = dst_v[pl.ds(off, 16)]
        rl = dv * N + sv
        f = plsc.load_gather(table_v, [rl])
        fc = jnp.clip(f, 0, E - 1)
        lf = plsc.load_gather(lin_v, [fc])
        valid = (f == fc) & (lf == rl)
        idx_v[pl.ds(i * 16, 16)] = jnp.where(valid, f + 1, 0)
        return carry

    lax.fori_loop(0, EPW // 16, resolve_body, 0)

    pltpu.async_copy(efpad_hbm.at[idx_v], rows_v, sem).wait()
    pltpu.sync_copy(idx_v, rp1_hbm.at[pl.ds(base, EPW)])
    pltpu.sync_copy(rows_v, revfeat_hbm.at[pl.ds(base, EPW)])


def _sc_rev(src, dst, efpad):
    mesh = plsc.VectorSubcoreMesh(core_axis_name="c", subcore_axis_name="s")
    fn = pl.kernel(
        _sc_rev_body,
        out_type=[
            jax.ShapeDtypeStruct((E,), jnp.int32),
            jax.ShapeDtypeStruct((E, D), jnp.float32),
        ],
        mesh=mesh,
        scratch_types=[
            pltpu.VMEM((E,), jnp.int32),
            pltpu.VMEM((E,), jnp.int32),
            pltpu.VMEM((E,), jnp.int32),
            pltpu.VMEM((N * N,), jnp.int32),
            pltpu.VMEM((EPW,), jnp.int32),
            pltpu.VMEM((EPW, D), jnp.float32),
            pltpu.SemaphoreType.DMA,
        ],
        compiler_params=pltpu.CompilerParams(needs_layout_passes=False),
    )
    return fn(src, dst, efpad)


def kernel(node_features, edge_features, edge_index, P_weights, Q_weights,
           pool_W, pool_b, W_weight):
    src = edge_index[:, 0]
    dst = edge_index[:, 1]
    efpad = jnp.concatenate(
        [jnp.zeros((1, D), jnp.float32), edge_features], axis=0)
    rp1, revfeat = _sc_rev(src, dst, efpad)

    src3 = src.reshape(NBLK, 1, EBLK)
    dst3 = dst.reshape(NBLK, 1, EBLK)
    rp13 = rp1.reshape(NBLK, 1, EBLK)
    pool_b2 = pool_b.reshape(1, D)

    s_sum = _pair_pool(node_features, edge_features, revfeat, src3, dst3,
                       rp13, P_weights, pool_W, pool_b2)
    qf, wf = _qf_wf(node_features, edge_features, src3, dst3, s_sum,
                    Q_weights, pool_b2, W_weight)
    return (qf, wf)


# speedup vs baseline: 1.8587x; 1.0019x over previous
import functools

import jax
import jax.numpy as jnp
from jax import lax
from jax.experimental import pallas as pl
from jax.experimental.pallas import tpu as pltpu
from jax.experimental.pallas import tpu_sc as plsc

N = 256
E = 8192
D = 128
EBLK = 512
NBLK = E // EBLK
NWORK = 32
EPW = E // NWORK


def _relu(x):
    return jnp.maximum(x, 0.0)


def _dot(a, b):
    return jax.lax.dot_general(a, b, (((1,), (0,)), ((), ())),
                               preferred_element_type=jnp.float32)


def _dot_t(a, b):
    return jax.lax.dot_general(a, b, (((1,), (1,)), ((), ())),
                               preferred_element_type=jnp.float32)


def _dot_c0(a, b):
    return jax.lax.dot_general(a, b, (((0,), (0,)), ((), ())),
                               preferred_element_type=jnp.float32)


def _k1_body(nf_ref, ef_ref, rvf_ref, src_ref, dst_ref, rp1_ref,
             pw_ref, poolw_ref, poolb_ref, s_out, nodep1_s, sacc_s):
    i = pl.program_id(0)

    @pl.when(i == 0)
    def _init():
        nodep1_s[...] = _dot(nf_ref[...], pw_ref[0:D, :])
        sacc_s[...] = jnp.zeros((N, D), jnp.float32)

    src_row = src_ref[0]
    dst_row = dst_ref[0]
    rp1_row = rp1_ref[0]
    iota_n = lax.broadcasted_iota(jnp.int32, (N, EBLK), 0)
    ohs = (iota_n == src_row).astype(jnp.float32)
    ohd = (iota_n == dst_row).astype(jnp.float32)
    maskb = (rp1_row == 0).astype(jnp.float32)
    ohd_m = ohd * maskb

    ef_b = ef_ref[...]
    rv_b = rvf_ref[...]
    p2 = pw_ref[D:2 * D, :]
    p3 = pw_ref[2 * D:3 * D, :]
    a_f = _dot(ef_b, p2)
    b_f = _dot(ef_b, p3)
    r_f = _dot(rv_b, p3)
    n_s = _dot_c0(ohs, nodep1_s[...])
    n_d = _dot_c0(ohd, nodep1_s[...])

    poolb = poolb_ref[...]
    relub = _relu(poolb)
    h1a = _relu(n_s + a_f + r_f)
    h2a = _relu(_dot_t(h1a, poolw_ref[...]) + poolb)
    h1b = _relu(n_d + b_f)
    h2b = _relu(_dot_t(h1b, poolw_ref[...]) + poolb)
    ca = h2a - relub
    cb = h2b - relub
    sacc_s[...] += _dot(ohs, ca) + _dot(ohd_m, cb)

    @pl.when(i == NBLK - 1)
    def _fin():
        s_out[...] = sacc_s[...]


def _k2_body(nf_ref, ef_ref, src_ref, dst_ref, s_ref, qw_ref, poolb_ref,
             ww_ref, qf_out, wf_out, qf_s, qw2_s, qw3_s):
    i = pl.program_id(0)

    @pl.when(i == 0)
    def _init():
        relub = _relu(poolb_ref[...])
        pf = relub + s_ref[...] * (1.0 / (N - 1))
        qf = _relu(_dot(nf_ref[...], qw_ref[0:D, :]) +
                   _dot(pf, qw_ref[D:2 * D, :]))
        qf_s[...] = qf
        qf_out[...] = qf
        qw2_s[...] = _dot(qf, ww_ref[D:2 * D, :])
        qw3_s[...] = _dot(qf, ww_ref[2 * D:3 * D, :])

    src_row = src_ref[0]
    dst_row = dst_ref[0]
    iota_n = lax.broadcasted_iota(jnp.int32, (N, EBLK), 0)
    ohs = (iota_n == src_row).astype(jnp.float32)
    ohd = (iota_n == dst_row).astype(jnp.float32)
    wf_out[...] = _relu(_dot(ef_ref[...], ww_ref[0:D, :]) +
                        _dot_c0(ohs, qw2_s[...]) +
                        _dot_c0(ohd, qw3_s[...]))


def _pair_pool(nf, ef, revfeat, src3, dst3, rp13, P_weights, pool_W, pool_b):
    return pl.pallas_call(
        _k1_body,
        grid=(NBLK,),
        in_specs=[
            pl.BlockSpec((N, D), lambda i: (0, 0)),
            pl.BlockSpec((EBLK, D), lambda i: (i, 0)),
            pl.BlockSpec((EBLK, D), lambda i: (i, 0)),
            pl.BlockSpec((1, 1, EBLK), lambda i: (i, 0, 0)),
            pl.BlockSpec((1, 1, EBLK), lambda i: (i, 0, 0)),
            pl.BlockSpec((1, 1, EBLK), lambda i: (i, 0, 0)),
            pl.BlockSpec((3 * D, D), lambda i: (0, 0)),
            pl.BlockSpec((D, D), lambda i: (0, 0)),
            pl.BlockSpec((1, D), lambda i: (0, 0)),
        ],
        out_specs=pl.BlockSpec((N, D), lambda i: (0, 0)),
        out_shape=jax.ShapeDtypeStruct((N, D), jnp.float32),
        scratch_shapes=[
            pltpu.VMEM((N, D), jnp.float32),
            pltpu.VMEM((N, D), jnp.float32),
        ],
    )(nf, ef, revfeat, src3, dst3, rp13, P_weights, pool_W, pool_b)


def _qf_wf(nf, ef, src3, dst3, s_sum, Q_weights, pool_b, W_weight):
    return pl.pallas_call(
        _k2_body,
        grid=(NBLK,),
        in_specs=[
            pl.BlockSpec((N, D), lambda i: (0, 0)),
            pl.BlockSpec((EBLK, D), lambda i: (i, 0)),
            pl.BlockSpec((1, 1, EBLK), lambda i: (i, 0, 0)),
            pl.BlockSpec((1, 1, EBLK), lambda i: (i, 0, 0)),
            pl.BlockSpec((N, D), lambda i: (0, 0)),
            pl.BlockSpec((2 * D, D), lambda i: (0, 0)),
            pl.BlockSpec((1, D), lambda i: (0, 0)),
            pl.BlockSpec((3 * D, D), lambda i: (0, 0)),
        ],
        out_specs=[
            pl.BlockSpec((N, D), lambda i: (0, 0)),
            pl.BlockSpec((EBLK, D), lambda i: (i, 0)),
        ],
        out_shape=[
            jax.ShapeDtypeStruct((N, D), jnp.float32),
            jax.ShapeDtypeStruct((E, D), jnp.float32),
        ],
        scratch_shapes=[
            pltpu.VMEM((N, D), jnp.float32),
            pltpu.VMEM((N, D), jnp.float32),
            pltpu.VMEM((N, D), jnp.float32),
        ],
    )(nf, ef, src3, dst3, s_sum, Q_weights, pool_b, W_weight)


def _sc_rev_body(src_hbm, dst_hbm, efpad_hbm, rp1_hbm, revfeat_hbm,
                 src_v, dst_v, lin_v, table_v, idx_v, rows_v, sem):
    cid = lax.axis_index("c")
    sid = lax.axis_index("s")
    wid = sid * 2 + cid

    pltpu.sync_copy(src_hbm, src_v)
    pltpu.sync_copy(dst_hbm, dst_v)
    iota = lax.iota(jnp.int32, 16)

    def scatter_body(i, carry):
        sv = src_v[pl.ds(i * 16, 16)]
        dv = dst_v[pl.ds(i * 16, 16)]
        linv = sv * N + dv
        lin_v[pl.ds(i * 16, 16)] = linv
        plsc.store_scatter(table_v, [linv], iota + i * 16)
        return carry

    lax.fori_loop(0, E // 16, scatter_body, 0)

    base = wid * EPW

    def resolve_body(i, carry):
        off = base + i * 16
        sv = src_v[pl.ds(off, 16)]
        dv = dst_v[pl.ds(off, 16)]
        rl = dv * N + sv
        f = plsc.load_gather(table_v, [rl])
        fc = jnp.clip(f, 0, E - 1)
        lf = plsc.load_gather(lin_v, [fc])
        valid = (f == fc) & (lf == rl)
        idx_v[pl.ds(i * 16, 16)] = jnp.where(valid, f + 1, 0)
        return carry

    lax.fori_loop(0, EPW // 16, resolve_body, 0)

    chunk = 16
    descs = []
    for j in range(EPW // chunk):
        descs.append(pltpu.async_copy(
            efpad_hbm.at[idx_v.at[pl.ds(j * chunk, chunk)]],
            rows_v.at[pl.ds(j * chunk, chunk)], sem))
    for d in descs:
        d.wait()
    pltpu.sync_copy(idx_v, rp1_hbm.at[pl.ds(base, EPW)])
    pltpu.sync_copy(rows_v, revfeat_hbm.at[pl.ds(base, EPW)])


def _sc_rev(src, dst, efpad):
    mesh = plsc.VectorSubcoreMesh(core_axis_name="c", subcore_axis_name="s")
    fn = pl.kernel(
        _sc_rev_body,
        out_type=[
            jax.ShapeDtypeStruct((E,), jnp.int32),
            jax.ShapeDtypeStruct((E, D), jnp.float32),
        ],
        mesh=mesh,
        scratch_types=[
            pltpu.VMEM((E,), jnp.int32),
            pltpu.VMEM((E,), jnp.int32),
            pltpu.VMEM((E,), jnp.int32),
            pltpu.VMEM((N * N,), jnp.int32),
            pltpu.VMEM((EPW,), jnp.int32),
            pltpu.VMEM((EPW, D), jnp.float32),
            pltpu.SemaphoreType.DMA,
        ],
        compiler_params=pltpu.CompilerParams(needs_layout_passes=False),
    )
    return fn(src, dst, efpad)


def kernel(node_features, edge_features, edge_index, P_weights, Q_weights,
           pool_W, pool_b, W_weight):
    src = edge_index[:, 0]
    dst = edge_index[:, 1]
    efpad = jnp.concatenate(
        [jnp.zeros((1, D), jnp.float32), edge_features], axis=0)
    rp1, revfeat = _sc_rev(src, dst, efpad)

    src3 = src.reshape(NBLK, 1, EBLK)
    dst3 = dst.reshape(NBLK, 1, EBLK)
    rp13 = rp1.reshape(NBLK, 1, EBLK)
    pool_b2 = pool_b.reshape(1, D)

    s_sum = _pair_pool(node_features, edge_features, revfeat, src3, dst3,
                       rp13, P_weights, pool_W, pool_b2)
    qf, wf = _qf_wf(node_features, edge_features, src3, dst3, s_sum,
                    Q_weights, pool_b2, W_weight)
    return (qf, wf)


# speedup vs baseline: 1.8593x; 1.0003x over previous
import functools

import jax
import jax.numpy as jnp
from jax import lax
from jax.experimental import pallas as pl
from jax.experimental.pallas import tpu as pltpu
from jax.experimental.pallas import tpu_sc as plsc

N = 256
E = 8192
D = 128
EBLK = 512
NBLK = E // EBLK
NWORK = 32
EPW = E // NWORK


def _relu(x):
    return jnp.maximum(x, 0.0)


def _dot(a, b):
    return jax.lax.dot_general(a, b, (((1,), (0,)), ((), ())),
                               preferred_element_type=jnp.float32)


def _dot_t(a, b):
    return jax.lax.dot_general(a, b, (((1,), (1,)), ((), ())),
                               preferred_element_type=jnp.float32)


def _dot_c0(a, b):
    return jax.lax.dot_general(a, b, (((0,), (0,)), ((), ())),
                               preferred_element_type=jnp.float32)


def _k1_body(nf_ref, ef_ref, rvf_ref, src_ref, dst_ref, rp1_ref,
             pw_ref, poolw_ref, poolb_ref, s_out, nodep1_s, sacc_s):
    i = pl.program_id(0)

    @pl.when(i == 0)
    def _init():
        nodep1_s[...] = _dot(nf_ref[...], pw_ref[0:D, :])
        sacc_s[...] = jnp.zeros((N, D), jnp.float32)

    src_row = src_ref[0]
    dst_row = dst_ref[0]
    rp1_row = rp1_ref[0]
    iota_n = lax.broadcasted_iota(jnp.int32, (N, EBLK), 0)
    ohs = (iota_n == src_row).astype(jnp.float32)
    ohd = (iota_n == dst_row).astype(jnp.float32)
    maskb = (rp1_row == 0).astype(jnp.float32)
    ohd_m = ohd * maskb

    ef_b = ef_ref[...]
    rv_b = rvf_ref[...]
    p2 = pw_ref[D:2 * D, :]
    p3 = pw_ref[2 * D:3 * D, :]
    a_f = _dot(ef_b, p2)
    b_f = _dot(ef_b, p3)
    r_f = _dot(rv_b, p3)
    n_s = _dot_c0(ohs, nodep1_s[...])
    n_d = _dot_c0(ohd, nodep1_s[...])

    poolb = poolb_ref[...]
    relub = _relu(poolb)
    h1a = _relu(n_s + a_f + r_f)
    h2a = _relu(_dot_t(h1a, poolw_ref[...]) + poolb)
    h1b = _relu(n_d + b_f)
    h2b = _relu(_dot_t(h1b, poolw_ref[...]) + poolb)
    ca = h2a - relub
    cb = h2b - relub
    sacc_s[...] += _dot(ohs, ca) + _dot(ohd_m, cb)

    @pl.when(i == NBLK - 1)
    def _fin():
        s_out[...] = sacc_s[...]


def _k2_body(nf_ref, ef_ref, src_ref, dst_ref, s_ref, qw_ref, poolb_ref,
             ww_ref, qf_out, wf_out, qf_s, qw2_s, qw3_s):
    i = pl.program_id(0)

    @pl.when(i == 0)
    def _init():
        relub = _relu(poolb_ref[...])
        pf = relub + s_ref[...] * (1.0 / (N - 1))
        qf = _relu(_dot(nf_ref[...], qw_ref[0:D, :]) +
                   _dot(pf, qw_ref[D:2 * D, :]))
        qf_s[...] = qf
        qf_out[...] = qf
        qw2_s[...] = _dot(qf, ww_ref[D:2 * D, :])
        qw3_s[...] = _dot(qf, ww_ref[2 * D:3 * D, :])

    src_row = src_ref[0]
    dst_row = dst_ref[0]
    iota_n = lax.broadcasted_iota(jnp.int32, (N, EBLK), 0)
    ohs = (iota_n == src_row).astype(jnp.float32)
    ohd = (iota_n == dst_row).astype(jnp.float32)
    wf_out[...] = _relu(_dot(ef_ref[...], ww_ref[0:D, :]) +
                        _dot_c0(ohs, qw2_s[...]) +
                        _dot_c0(ohd, qw3_s[...]))


def _pair_pool(nf, ef, revfeat, src3, dst3, rp13, P_weights, pool_W, pool_b):
    return pl.pallas_call(
        _k1_body,
        grid=(NBLK,),
        in_specs=[
            pl.BlockSpec((N, D), lambda i: (0, 0)),
            pl.BlockSpec((EBLK, D), lambda i: (i, 0)),
            pl.BlockSpec((EBLK, D), lambda i: (i, 0)),
            pl.BlockSpec((1, 1, EBLK), lambda i: (i, 0, 0)),
            pl.BlockSpec((1, 1, EBLK), lambda i: (i, 0, 0)),
            pl.BlockSpec((1, 1, EBLK), lambda i: (i, 0, 0)),
            pl.BlockSpec((3 * D, D), lambda i: (0, 0)),
            pl.BlockSpec((D, D), lambda i: (0, 0)),
            pl.BlockSpec((1, D), lambda i: (0, 0)),
        ],
        out_specs=pl.BlockSpec((N, D), lambda i: (0, 0)),
        out_shape=jax.ShapeDtypeStruct((N, D), jnp.float32),
        scratch_shapes=[
            pltpu.VMEM((N, D), jnp.float32),
            pltpu.VMEM((N, D), jnp.float32),
        ],
    )(nf, ef, revfeat, src3, dst3, rp13, P_weights, pool_W, pool_b)


def _qf_wf(nf, ef, src3, dst3, s_sum, Q_weights, pool_b, W_weight):
    return pl.pallas_call(
        _k2_body,
        grid=(NBLK,),
        in_specs=[
            pl.BlockSpec((N, D), lambda i: (0, 0)),
            pl.BlockSpec((EBLK, D), lambda i: (i, 0)),
            pl.BlockSpec((1, 1, EBLK), lambda i: (i, 0, 0)),
            pl.BlockSpec((1, 1, EBLK), lambda i: (i, 0, 0)),
            pl.BlockSpec((N, D), lambda i: (0, 0)),
            pl.BlockSpec((2 * D, D), lambda i: (0, 0)),
            pl.BlockSpec((1, D), lambda i: (0, 0)),
            pl.BlockSpec((3 * D, D), lambda i: (0, 0)),
        ],
        out_specs=[
            pl.BlockSpec((N, D), lambda i: (0, 0)),
            pl.BlockSpec((EBLK, D), lambda i: (i, 0)),
        ],
        out_shape=[
            jax.ShapeDtypeStruct((N, D), jnp.float32),
            jax.ShapeDtypeStruct((E, D), jnp.float32),
        ],
        scratch_shapes=[
            pltpu.VMEM((N, D), jnp.float32),
            pltpu.VMEM((N, D), jnp.float32),
            pltpu.VMEM((N, D), jnp.float32),
        ],
    )(nf, ef, src3, dst3, s_sum, Q_weights, pool_b, W_weight)


def _sc_rev_body(src_hbm, dst_hbm, efpad_hbm, rp1_hbm, revfeat_hbm,
                 src_v, dst_v, lin_v, table_v, idx_v, rows_v, sem):
    cid = lax.axis_index("c")
    sid = lax.axis_index("s")
    wid = sid * 2 + cid

    with jax.named_scope("sc_copy_in"):
        pltpu.sync_copy(src_hbm, src_v)
        pltpu.sync_copy(dst_hbm, dst_v)
    iota = lax.iota(jnp.int32, 16)

    def scatter_body(i, carry):
        sv = src_v[pl.ds(i * 16, 16)]
        dv = dst_v[pl.ds(i * 16, 16)]
        linv = sv * N + dv
        lin_v[pl.ds(i * 16, 16)] = linv
        plsc.store_scatter(table_v, [linv], iota + i * 16)
        return carry

    with jax.named_scope("sc_scatter"):
        lax.fori_loop(0, E // 16, scatter_body, 0)

    base = wid * EPW

    def resolve_body(i, carry):
        off = base + i * 16
        sv = src_v[pl.ds(off, 16)]
        dv = dst_v[pl.ds(off, 16)]
        rl = dv * N + sv
        f = plsc.load_gather(table_v, [rl])
        fc = jnp.clip(f, 0, E - 1)
        lf = plsc.load_gather(lin_v, [fc])
        valid = (f == fc) & (lf == rl)
        idx_v[pl.ds(i * 16, 16)] = jnp.where(valid, f + 1, 0)
        return carry

    with jax.named_scope("sc_resolve"):
        lax.fori_loop(0, EPW // 16, resolve_body, 0)

    with jax.named_scope("sc_gather"):
        chunk = 16
        descs = []
        for j in range(EPW // chunk):
            descs.append(pltpu.async_copy(
                efpad_hbm.at[idx_v.at[pl.ds(j * chunk, chunk)]],
                rows_v.at[pl.ds(j * chunk, chunk)], sem))
        for d in descs:
            d.wait()
    with jax.named_scope("sc_write_out"):
        pltpu.sync_copy(idx_v, rp1_hbm.at[pl.ds(base, EPW)])
        pltpu.sync_copy(rows_v, revfeat_hbm.at[pl.ds(base, EPW)])


def _sc_rev(src, dst, efpad):
    mesh = plsc.VectorSubcoreMesh(core_axis_name="c", subcore_axis_name="s")
    fn = pl.kernel(
        _sc_rev_body,
        out_type=[
            jax.ShapeDtypeStruct((E,), jnp.int32),
            jax.ShapeDtypeStruct((E, D), jnp.float32),
        ],
        mesh=mesh,
        scratch_types=[
            pltpu.VMEM((E,), jnp.int32),
            pltpu.VMEM((E,), jnp.int32),
            pltpu.VMEM((E,), jnp.int32),
            pltpu.VMEM((N * N,), jnp.int32),
            pltpu.VMEM((EPW,), jnp.int32),
            pltpu.VMEM((EPW, D), jnp.float32),
            pltpu.SemaphoreType.DMA,
        ],
        compiler_params=pltpu.CompilerParams(needs_layout_passes=False),
    )
    return fn(src, dst, efpad)


def kernel(node_features, edge_features, edge_index, P_weights, Q_weights,
           pool_W, pool_b, W_weight):
    src = edge_index[:, 0]
    dst = edge_index[:, 1]
    efpad = jnp.concatenate(
        [jnp.zeros((1, D), jnp.float32), edge_features], axis=0)
    rp1, revfeat = _sc_rev(src, dst, efpad)

    src3 = src.reshape(NBLK, 1, EBLK)
    dst3 = dst.reshape(NBLK, 1, EBLK)
    rp13 = rp1.reshape(NBLK, 1, EBLK)
    pool_b2 = pool_b.reshape(1, D)

    s_sum = _pair_pool(node_features, edge_features, revfeat, src3, dst3,
                       rp13, P_weights, pool_W, pool_b2)
    qf, wf = _qf_wf(node_features, edge_features, src3, dst3, s_sum,
                    Q_weights, pool_b2, W_weight)
    return (qf, wf)


# speedup vs baseline: 7.3531x; 3.9547x over previous
import functools

import jax
import jax.numpy as jnp
from jax import lax
from jax.experimental import pallas as pl
from jax.experimental.pallas import tpu as pltpu
from jax.experimental.pallas import tpu_sc as plsc

N = 256
E = 8192
D = 128
EBLK = 512
NBLK = E // EBLK
NWORK = 32
EPW = E // NWORK


def _relu(x):
    return jnp.maximum(x, 0.0)


def _dot(a, b):
    return jax.lax.dot_general(a, b, (((1,), (0,)), ((), ())),
                               preferred_element_type=jnp.float32)


def _dot_t(a, b):
    return jax.lax.dot_general(a, b, (((1,), (1,)), ((), ())),
                               preferred_element_type=jnp.float32)


def _dot_c0(a, b):
    return jax.lax.dot_general(a, b, (((0,), (0,)), ((), ())),
                               preferred_element_type=jnp.float32)


def _k1_body(nf_ref, ef_ref, rvf_ref, src_ref, dst_ref, rp1_ref,
             pw_ref, poolw_ref, poolb_ref, s_out, nodep1_s, sacc_s):
    i = pl.program_id(0)

    @pl.when(i == 0)
    def _init():
        nodep1_s[...] = _dot(nf_ref[...], pw_ref[0:D, :])
        sacc_s[...] = jnp.zeros((N, D), jnp.float32)

    src_row = src_ref[0]
    dst_row = dst_ref[0]
    rp1_row = rp1_ref[0]
    iota_n = lax.broadcasted_iota(jnp.int32, (N, EBLK), 0)
    ohs = (iota_n == src_row).astype(jnp.float32)
    ohd = (iota_n == dst_row).astype(jnp.float32)
    maskb = (rp1_row == 0).astype(jnp.float32)
    ohd_m = ohd * maskb

    ef_b = ef_ref[...]
    rv_b = rvf_ref[...]
    p2 = pw_ref[D:2 * D, :]
    p3 = pw_ref[2 * D:3 * D, :]
    a_f = _dot(ef_b, p2)
    b_f = _dot(ef_b, p3)
    r_f = _dot(rv_b, p3)
    n_s = _dot_c0(ohs, nodep1_s[...])
    n_d = _dot_c0(ohd, nodep1_s[...])

    poolb = poolb_ref[...]
    relub = _relu(poolb)
    h1a = _relu(n_s + a_f + r_f)
    h2a = _relu(_dot_t(h1a, poolw_ref[...]) + poolb)
    h1b = _relu(n_d + b_f)
    h2b = _relu(_dot_t(h1b, poolw_ref[...]) + poolb)
    ca = h2a - relub
    cb = h2b - relub
    sacc_s[...] += _dot(ohs, ca) + _dot(ohd_m, cb)

    @pl.when(i == NBLK - 1)
    def _fin():
        s_out[...] = sacc_s[...]


def _k2_body(nf_ref, ef_ref, src_ref, dst_ref, s_ref, qw_ref, poolb_ref,
             ww_ref, qf_out, wf_out, qf_s, qw2_s, qw3_s):
    i = pl.program_id(0)

    @pl.when(i == 0)
    def _init():
        relub = _relu(poolb_ref[...])
        pf = relub + s_ref[...] * (1.0 / (N - 1))
        qf = _relu(_dot(nf_ref[...], qw_ref[0:D, :]) +
                   _dot(pf, qw_ref[D:2 * D, :]))
        qf_s[...] = qf
        qf_out[...] = qf
        qw2_s[...] = _dot(qf, ww_ref[D:2 * D, :])
        qw3_s[...] = _dot(qf, ww_ref[2 * D:3 * D, :])

    src_row = src_ref[0]
    dst_row = dst_ref[0]
    iota_n = lax.broadcasted_iota(jnp.int32, (N, EBLK), 0)
    ohs = (iota_n == src_row).astype(jnp.float32)
    ohd = (iota_n == dst_row).astype(jnp.float32)
    wf_out[...] = _relu(_dot(ef_ref[...], ww_ref[0:D, :]) +
                        _dot_c0(ohs, qw2_s[...]) +
                        _dot_c0(ohd, qw3_s[...]))


def _pair_pool(nf, ef, revfeat, src3, dst3, rp13, P_weights, pool_W, pool_b):
    return pl.pallas_call(
        _k1_body,
        grid=(NBLK,),
        in_specs=[
            pl.BlockSpec((N, D), lambda i: (0, 0)),
            pl.BlockSpec((EBLK, D), lambda i: (i, 0)),
            pl.BlockSpec((EBLK, D), lambda i: (i, 0)),
            pl.BlockSpec((1, 1, EBLK), lambda i: (i, 0, 0)),
            pl.BlockSpec((1, 1, EBLK), lambda i: (i, 0, 0)),
            pl.BlockSpec((1, 1, EBLK), lambda i: (i, 0, 0)),
            pl.BlockSpec((3 * D, D), lambda i: (0, 0)),
            pl.BlockSpec((D, D), lambda i: (0, 0)),
            pl.BlockSpec((1, D), lambda i: (0, 0)),
        ],
        out_specs=pl.BlockSpec((N, D), lambda i: (0, 0)),
        out_shape=jax.ShapeDtypeStruct((N, D), jnp.float32),
        scratch_shapes=[
            pltpu.VMEM((N, D), jnp.float32),
            pltpu.VMEM((N, D), jnp.float32),
        ],
    )(nf, ef, revfeat, src3, dst3, rp13, P_weights, pool_W, pool_b)


def _qf_wf(nf, ef, src3, dst3, s_sum, Q_weights, pool_b, W_weight):
    return pl.pallas_call(
        _k2_body,
        grid=(NBLK,),
        in_specs=[
            pl.BlockSpec((N, D), lambda i: (0, 0)),
            pl.BlockSpec((EBLK, D), lambda i: (i, 0)),
            pl.BlockSpec((1, 1, EBLK), lambda i: (i, 0, 0)),
            pl.BlockSpec((1, 1, EBLK), lambda i: (i, 0, 0)),
            pl.BlockSpec((N, D), lambda i: (0, 0)),
            pl.BlockSpec((2 * D, D), lambda i: (0, 0)),
            pl.BlockSpec((1, D), lambda i: (0, 0)),
            pl.BlockSpec((3 * D, D), lambda i: (0, 0)),
        ],
        out_specs=[
            pl.BlockSpec((N, D), lambda i: (0, 0)),
            pl.BlockSpec((EBLK, D), lambda i: (i, 0)),
        ],
        out_shape=[
            jax.ShapeDtypeStruct((N, D), jnp.float32),
            jax.ShapeDtypeStruct((E, D), jnp.float32),
        ],
        scratch_shapes=[
            pltpu.VMEM((N, D), jnp.float32),
            pltpu.VMEM((N, D), jnp.float32),
            pltpu.VMEM((N, D), jnp.float32),
        ],
    )(nf, ef, src3, dst3, s_sum, Q_weights, pool_b, W_weight)


_CHUNK = 1024


def _sc_rev_body(src_hbm, dst_hbm, efpad_hbm, rp1_hbm, revfeat_hbm,
                 table_v, srcc_v, dstc_v, osrc_v, odst_v, idx_v, rows_v,
                 ef_sp, sem, sem_stage):
    cid = lax.axis_index("c")
    sid = lax.axis_index("s")
    wid = sid * 2 + cid

    @pl.when(sid == 0)
    def _stage():
        pltpu.async_copy(efpad_hbm, ef_sp, sem_stage)

    with jax.named_scope("sc_zero"):
        zeros16 = jnp.zeros((16,), jnp.int32)

        def zero_body(i, carry):
            table_v[pl.ds(i * 16, 16)] = zeros16
            return carry

        lax.fori_loop(0, (N * N // 2) // 16, zero_body, 0)

    iota = lax.iota(jnp.int32, 16)

    with jax.named_scope("sc_scatter"):
        for c in range(E // _CHUNK):
            pltpu.sync_copy(src_hbm.at[pl.ds(c * _CHUNK, _CHUNK)], srcc_v)
            pltpu.sync_copy(dst_hbm.at[pl.ds(c * _CHUNK, _CHUNK)], dstc_v)

            def scatter_body(i, carry):
                sv = srcc_v[pl.ds(i * 16, 16)]
                dv = dstc_v[pl.ds(i * 16, 16)]
                linv = sv * N + dv
                val = (iota + (c * _CHUNK + 1) + i * 16) << ((linv & 1) * 16)
                plsc.addupdate_scatter(table_v, [linv >> 1], val)
                return carry

            lax.fori_loop(0, _CHUNK // 16, scatter_body, 0)

    base = wid * EPW
    with jax.named_scope("sc_resolve"):
        pltpu.sync_copy(src_hbm.at[pl.ds(base, EPW)], osrc_v)
        pltpu.sync_copy(dst_hbm.at[pl.ds(base, EPW)], odst_v)

        def resolve_body(i, carry):
            rl = odst_v[pl.ds(i * 16, 16)] * N + osrc_v[pl.ds(i * 16, 16)]
            w = plsc.load_gather(table_v, [rl >> 1])
            rp1 = (w >> ((rl & 1) * 16)) & 0xFFFF
            idx_v[pl.ds(i * 16, 16)] = rp1
            return carry

        lax.fori_loop(0, EPW // 16, resolve_body, 0)

    @pl.when(sid == 0)
    def _stage_wait():
        pltpu.make_async_copy(efpad_hbm, ef_sp, sem_stage).wait()

    plsc.subcore_barrier()

    half = EPW // 2
    with jax.named_scope("sc_gather"):
        g0 = pltpu.async_copy(ef_sp.at[idx_v.at[pl.ds(0, half)]],
                              rows_v, sem)
        pltpu.sync_copy(idx_v, rp1_hbm.at[pl.ds(base, EPW)])
        g0.wait()
        pltpu.sync_copy(rows_v, revfeat_hbm.at[pl.ds(base, half)])
        g1 = pltpu.async_copy(ef_sp.at[idx_v.at[pl.ds(half, half)]],
                              rows_v, sem)
        g1.wait()
        pltpu.sync_copy(rows_v, revfeat_hbm.at[pl.ds(base + half, half)])


def _sc_rev(src, dst, efpad):
    mesh = plsc.VectorSubcoreMesh(core_axis_name="c", subcore_axis_name="s")
    fn = pl.kernel(
        _sc_rev_body,
        out_type=[
            jax.ShapeDtypeStruct((E,), jnp.int32),
            jax.ShapeDtypeStruct((E, D), jnp.float32),
        ],
        mesh=mesh,
        scratch_types=[
            pltpu.VMEM((N * N // 2,), jnp.int32),
            pltpu.VMEM((_CHUNK,), jnp.int32),
            pltpu.VMEM((_CHUNK,), jnp.int32),
            pltpu.VMEM((EPW,), jnp.int32),
            pltpu.VMEM((EPW,), jnp.int32),
            pltpu.VMEM((EPW,), jnp.int32),
            pltpu.VMEM((EPW // 2, D), jnp.float32),
            pltpu.VMEM_SHARED((E + 1, D), jnp.float32),
            pltpu.SemaphoreType.DMA,
            pltpu.SemaphoreType.DMA,
        ],
        compiler_params=pltpu.CompilerParams(needs_layout_passes=False),
    )
    return fn(src, dst, efpad)


def kernel(node_features, edge_features, edge_index, P_weights, Q_weights,
           pool_W, pool_b, W_weight):
    src = edge_index[:, 0]
    dst = edge_index[:, 1]
    efpad = jnp.concatenate(
        [jnp.zeros((1, D), jnp.float32), edge_features], axis=0)
    rp1, revfeat = _sc_rev(src, dst, efpad)

    src3 = src.reshape(NBLK, 1, EBLK)
    dst3 = dst.reshape(NBLK, 1, EBLK)
    rp13 = rp1.reshape(NBLK, 1, EBLK)
    pool_b2 = pool_b.reshape(1, D)

    s_sum = _pair_pool(node_features, edge_features, revfeat, src3, dst3,
                       rp13, P_weights, pool_W, pool_b2)
    qf, wf = _qf_wf(node_features, edge_features, src3, dst3, s_sum,
                    Q_weights, pool_b2, W_weight)
    return (qf, wf)


# speedup vs baseline: 8.4917x; 1.1548x over previous
import functools

import jax
import jax.numpy as jnp
from jax import lax
from jax.experimental import pallas as pl
from jax.experimental.pallas import tpu as pltpu
from jax.experimental.pallas import tpu_sc as plsc

N = 256
E = 8192
D = 128
EBLK = 512
NBLK = E // EBLK
NWORK = 32
EPW = E // NWORK


def _relu(x):
    return jnp.maximum(x, 0.0)


def _dot(a, b):
    return jax.lax.dot_general(a, b, (((1,), (0,)), ((), ())),
                               preferred_element_type=jnp.float32)


def _dot_t(a, b):
    return jax.lax.dot_general(a, b, (((1,), (1,)), ((), ())),
                               preferred_element_type=jnp.float32)


def _dot_c0(a, b):
    return jax.lax.dot_general(a, b, (((0,), (0,)), ((), ())),
                               preferred_element_type=jnp.float32)


def _k1_body(nf_ref, ef_ref, rvf_ref, src_ref, dst_ref, rp1_ref,
             pw_ref, poolw_ref, poolb_ref, s_out, nodep1_s, sacc_s):
    i = pl.program_id(0)

    @pl.when(i == 0)
    def _init():
        nodep1_s[...] = _dot(nf_ref[...], pw_ref[0:D, :])
        sacc_s[...] = jnp.zeros((N, D), jnp.float32)

    src_row = src_ref[0]
    dst_row = dst_ref[0]
    rp1_row = rp1_ref[0]
    iota_n = lax.broadcasted_iota(jnp.int32, (N, EBLK), 0)
    ohs = (iota_n == src_row).astype(jnp.float32)
    ohd = (iota_n == dst_row).astype(jnp.float32)
    maskb = (rp1_row == 0).astype(jnp.float32)
    ohd_m = ohd * maskb

    ef_b = ef_ref[...]
    rv_b = rvf_ref[...]
    p2 = pw_ref[D:2 * D, :]
    p3 = pw_ref[2 * D:3 * D, :]
    a_f = _dot(ef_b, p2)
    b_f = _dot(ef_b, p3)
    r_f = _dot(rv_b, p3)
    n_s = _dot_c0(ohs, nodep1_s[...])
    n_d = _dot_c0(ohd, nodep1_s[...])

    poolb = poolb_ref[...]
    relub = _relu(poolb)
    h1a = _relu(n_s + a_f + r_f)
    h2a = _relu(_dot_t(h1a, poolw_ref[...]) + poolb)
    h1b = _relu(n_d + b_f)
    h2b = _relu(_dot_t(h1b, poolw_ref[...]) + poolb)
    ca = h2a - relub
    cb = h2b - relub
    sacc_s[...] += _dot(ohs, ca) + _dot(ohd_m, cb)

    @pl.when(i == NBLK - 1)
    def _fin():
        s_out[...] = sacc_s[...]


def _k2_body(nf_ref, ef_ref, src_ref, dst_ref, s_ref, qw_ref, poolb_ref,
             ww_ref, qf_out, wf_out, qf_s, qw2_s, qw3_s):
    i = pl.program_id(0)

    @pl.when(i == 0)
    def _init():
        relub = _relu(poolb_ref[...])
        pf = relub + s_ref[...] * (1.0 / (N - 1))
        qf = _relu(_dot(nf_ref[...], qw_ref[0:D, :]) +
                   _dot(pf, qw_ref[D:2 * D, :]))
        qf_s[...] = qf
        qf_out[...] = qf
        qw2_s[...] = _dot(qf, ww_ref[D:2 * D, :])
        qw3_s[...] = _dot(qf, ww_ref[2 * D:3 * D, :])

    src_row = src_ref[0]
    dst_row = dst_ref[0]
    iota_n = lax.broadcasted_iota(jnp.int32, (N, EBLK), 0)
    ohs = (iota_n == src_row).astype(jnp.float32)
    ohd = (iota_n == dst_row).astype(jnp.float32)
    wf_out[...] = _relu(_dot(ef_ref[...], ww_ref[0:D, :]) +
                        _dot_c0(ohs, qw2_s[...]) +
                        _dot_c0(ohd, qw3_s[...]))


def _pair_pool(nf, ef, revfeat, src3, dst3, rp13, P_weights, pool_W, pool_b):
    return pl.pallas_call(
        _k1_body,
        grid=(NBLK,),
        in_specs=[
            pl.BlockSpec((N, D), lambda i: (0, 0)),
            pl.BlockSpec((EBLK, D), lambda i: (i, 0)),
            pl.BlockSpec((EBLK, D), lambda i: (i, 0)),
            pl.BlockSpec((1, 1, EBLK), lambda i: (i, 0, 0)),
            pl.BlockSpec((1, 1, EBLK), lambda i: (i, 0, 0)),
            pl.BlockSpec((1, 1, EBLK), lambda i: (i, 0, 0)),
            pl.BlockSpec((3 * D, D), lambda i: (0, 0)),
            pl.BlockSpec((D, D), lambda i: (0, 0)),
            pl.BlockSpec((1, D), lambda i: (0, 0)),
        ],
        out_specs=pl.BlockSpec((N, D), lambda i: (0, 0)),
        out_shape=jax.ShapeDtypeStruct((N, D), jnp.float32),
        scratch_shapes=[
            pltpu.VMEM((N, D), jnp.float32),
            pltpu.VMEM((N, D), jnp.float32),
        ],
    )(nf, ef, revfeat, src3, dst3, rp13, P_weights, pool_W, pool_b)


def _qf_wf(nf, ef, src3, dst3, s_sum, Q_weights, pool_b, W_weight):
    return pl.pallas_call(
        _k2_body,
        grid=(NBLK,),
        in_specs=[
            pl.BlockSpec((N, D), lambda i: (0, 0)),
            pl.BlockSpec((EBLK, D), lambda i: (i, 0)),
            pl.BlockSpec((1, 1, EBLK), lambda i: (i, 0, 0)),
            pl.BlockSpec((1, 1, EBLK), lambda i: (i, 0, 0)),
            pl.BlockSpec((N, D), lambda i: (0, 0)),
            pl.BlockSpec((2 * D, D), lambda i: (0, 0)),
            pl.BlockSpec((1, D), lambda i: (0, 0)),
            pl.BlockSpec((3 * D, D), lambda i: (0, 0)),
        ],
        out_specs=[
            pl.BlockSpec((N, D), lambda i: (0, 0)),
            pl.BlockSpec((EBLK, D), lambda i: (i, 0)),
        ],
        out_shape=[
            jax.ShapeDtypeStruct((N, D), jnp.float32),
            jax.ShapeDtypeStruct((E, D), jnp.float32),
        ],
        scratch_shapes=[
            pltpu.VMEM((N, D), jnp.float32),
            pltpu.VMEM((N, D), jnp.float32),
            pltpu.VMEM((N, D), jnp.float32),
        ],
    )(nf, ef, src3, dst3, s_sum, Q_weights, pool_b, W_weight)


_CHUNK = 1024


def _sc_rev_body(src_hbm, dst_hbm, ef_hbm, rp1_hbm, revfeat_hbm,
                 table_v, srcc_v, dstc_v, osrc_v, odst_v, idx_v, rows_v,
                 zrow_v, ef_sp, sem, sem_stage, sem_ping, sem_pong):
    cid = lax.axis_index("c")
    sid = lax.axis_index("s")
    wid = sid * 2 + cid

    @pl.when(sid == 0)
    def _stage():
        pltpu.async_copy(ef_hbm, ef_sp.at[pl.ds(1, E)], sem_stage)

    with jax.named_scope("sc_zero"):
        zeros16 = jnp.zeros((16,), jnp.int32)

        def zero_body(i, carry):
            for k in range(8):
                table_v[pl.ds(i * 128 + k * 16, 16)] = zeros16
            return carry

        lax.fori_loop(0, (N * N // 2) // 128, zero_body, 0)

    iota = lax.iota(jnp.int32, 16)
    nch = E // _CHUNK

    with jax.named_scope("sc_scatter"):
        halfc = _CHUNK // 2
        bufs = [(srcc_v.at[pl.ds(0, halfc)], dstc_v.at[pl.ds(0, halfc)],
                 sem_ping),
                (srcc_v.at[pl.ds(halfc, halfc)],
                 dstc_v.at[pl.ds(halfc, halfc)], sem_pong)]
        nch2 = E // halfc

        def start(c):
            sb, db, sm = bufs[c % 2]
            ds = pltpu.async_copy(src_hbm.at[pl.ds(c * halfc, halfc)], sb, sm)
            dd = pltpu.async_copy(dst_hbm.at[pl.ds(c * halfc, halfc)], db, sm)
            return ds, dd

        pend = start(0)
        for c in range(nch2):
            sb, db, _ = bufs[c % 2]
            for d in pend:
                d.wait()
            if c + 1 < nch2:
                pend = start(c + 1)

            def scatter_body(i, carry, sb=sb, db=db, c=c):
                for k in range(4):
                    off = i * 64 + k * 16
                    sv = sb[pl.ds(off, 16)]
                    dv = db[pl.ds(off, 16)]
                    linv = sv * N + dv
                    val = ((iota + (c * halfc + 1) + off)
                           << ((linv & 1) * 16))
                    plsc.addupdate_scatter(table_v, [linv >> 1], val)
                return carry

            lax.fori_loop(0, halfc // 64, scatter_body, 0)

    base = wid * EPW
    with jax.named_scope("sc_resolve"):
        pltpu.sync_copy(src_hbm.at[pl.ds(base, EPW)], osrc_v)
        pltpu.sync_copy(dst_hbm.at[pl.ds(base, EPW)], odst_v)

        def resolve_body(i, carry):
            rl = odst_v[pl.ds(i * 16, 16)] * N + osrc_v[pl.ds(i * 16, 16)]
            w = plsc.load_gather(table_v, [rl >> 1])
            rp1 = (w >> ((rl & 1) * 16)) & 0xFFFF
            idx_v[pl.ds(i * 16, 16)] = rp1
            return carry

        lax.fori_loop(0, EPW // 16, resolve_body, 0)

    @pl.when(sid == 0)
    def _stage_wait():
        zf = jnp.zeros((16,), jnp.float32)
        for k in range(8):
            zrow_v[pl.ds(k * 16, 16)] = zf
        pltpu.sync_copy(zrow_v, ef_sp.at[0])
        pltpu.make_async_copy(ef_hbm, ef_sp.at[pl.ds(1, E)],
                              sem_stage).wait()

    plsc.subcore_barrier()

    half = EPW // 2
    with jax.named_scope("sc_gather"):
        g0 = pltpu.async_copy(ef_sp.at[idx_v.at[pl.ds(0, half)]],
                              rows_v, sem)
        pltpu.sync_copy(idx_v, rp1_hbm.at[pl.ds(base, EPW)])
        g0.wait()
        pltpu.sync_copy(rows_v, revfeat_hbm.at[pl.ds(base, half)])
        g1 = pltpu.async_copy(ef_sp.at[idx_v.at[pl.ds(half, half)]],
                              rows_v, sem)
        g1.wait()
        pltpu.sync_copy(rows_v, revfeat_hbm.at[pl.ds(base + half, half)])


def _sc_rev(src, dst, ef):
    mesh = plsc.VectorSubcoreMesh(core_axis_name="c", subcore_axis_name="s")
    fn = pl.kernel(
        _sc_rev_body,
        out_type=[
            jax.ShapeDtypeStruct((E,), jnp.int32),
            jax.ShapeDtypeStruct((E, D), jnp.float32),
        ],
        mesh=mesh,
        scratch_types=[
            pltpu.VMEM((N * N // 2,), jnp.int32),
            pltpu.VMEM((_CHUNK,), jnp.int32),
            pltpu.VMEM((_CHUNK,), jnp.int32),
            pltpu.VMEM((EPW,), jnp.int32),
            pltpu.VMEM((EPW,), jnp.int32),
            pltpu.VMEM((EPW,), jnp.int32),
            pltpu.VMEM((EPW // 2, D), jnp.float32),
            pltpu.VMEM((D,), jnp.float32),
            pltpu.VMEM_SHARED((E + 1, D), jnp.float32),
            pltpu.SemaphoreType.DMA,
            pltpu.SemaphoreType.DMA,
            pltpu.SemaphoreType.DMA,
            pltpu.SemaphoreType.DMA,
        ],
        compiler_params=pltpu.CompilerParams(needs_layout_passes=False),
    )
    return fn(src, dst, ef)


def kernel(node_features, edge_features, edge_index, P_weights, Q_weights,
           pool_W, pool_b, W_weight):
    src = edge_index[:, 0]
    dst = edge_index[:, 1]
    rp1, revfeat = _sc_rev(src, dst, edge_features)

    src3 = src.reshape(NBLK, 1, EBLK)
    dst3 = dst.reshape(NBLK, 1, EBLK)
    rp13 = rp1.reshape(NBLK, 1, EBLK)
    pool_b2 = pool_b.reshape(1, D)

    s_sum = _pair_pool(node_features, edge_features, revfeat, src3, dst3,
                       rp13, P_weights, pool_W, pool_b2)
    qf, wf = _qf_wf(node_features, edge_features, src3, dst3, s_sum,
                    Q_weights, pool_b2, W_weight)
    return (qf, wf)


# speedup vs baseline: 8.6274x; 1.0160x over previous
import functools

import jax
import jax.numpy as jnp
from jax import lax
from jax.experimental import pallas as pl
from jax.experimental.pallas import tpu as pltpu
from jax.experimental.pallas import tpu_sc as plsc

N = 256
E = 8192
D = 128
EBLK = 512
NBLK = E // EBLK
NWORK = 32
EPW = E // NWORK


def _relu(x):
    return jnp.maximum(x, 0.0)


def _dot(a, b):
    return jax.lax.dot_general(a, b, (((1,), (0,)), ((), ())),
                               preferred_element_type=jnp.float32)


def _dot_t(a, b):
    return jax.lax.dot_general(a, b, (((1,), (1,)), ((), ())),
                               preferred_element_type=jnp.float32)


def _dot_c0(a, b):
    return jax.lax.dot_general(a, b, (((0,), (0,)), ((), ())),
                               preferred_element_type=jnp.float32)


def _k1a_body(nf_ref, ef_ref, src_ref, dst_ref, pw_ref, poolw_ref,
              poolb_ref, prea_out, h2b_out, nodep1_s):
    i = pl.program_id(0)

    @pl.when(i == 0)
    def _init():
        nodep1_s[...] = _dot(nf_ref[...], pw_ref[0:D, :])

    src_row = src_ref[0]
    dst_row = dst_ref[0]
    iota_n = lax.broadcasted_iota(jnp.int32, (N, EBLK), 0)
    ohs = (iota_n == src_row).astype(jnp.float32)
    ohd = (iota_n == dst_row).astype(jnp.float32)

    ef_b = ef_ref[...]
    p2 = pw_ref[D:2 * D, :]
    p3 = pw_ref[2 * D:3 * D, :]
    a_f = _dot(ef_b, p2)
    b_f = _dot(ef_b, p3)
    n_s = _dot_c0(ohs, nodep1_s[...])
    n_d = _dot_c0(ohd, nodep1_s[...])

    poolb = poolb_ref[...]
    prea_out[...] = n_s + a_f
    h1b = _relu(n_d + b_f)
    h2b_out[...] = _relu(_dot_t(h1b, poolw_ref[...]) + poolb)


def _k1b_body(rvf_ref, prea_ref, h2b_ref, src_ref, dst_ref, rp1_ref,
              pw_ref, poolw_ref, poolb_ref, s_out, sacc_s):
    i = pl.program_id(0)

    @pl.when(i == 0)
    def _init():
        sacc_s[...] = jnp.zeros((N, D), jnp.float32)

    src_row = src_ref[0]
    dst_row = dst_ref[0]
    rp1_row = rp1_ref[0]
    iota_n = lax.broadcasted_iota(jnp.int32, (N, EBLK), 0)
    ohs = (iota_n == src_row).astype(jnp.float32)
    ohd = (iota_n == dst_row).astype(jnp.float32)
    maskb = (rp1_row == 0).astype(jnp.float32)
    ohd_m = ohd * maskb

    rv_b = rvf_ref[...]
    p3 = pw_ref[2 * D:3 * D, :]
    r_f = _dot(rv_b, p3)

    poolb = poolb_ref[...]
    relub = _relu(poolb)
    h1a = _relu(prea_ref[...] + r_f)
    h2a = _relu(_dot_t(h1a, poolw_ref[...]) + poolb)
    ca = h2a - relub
    cb = h2b_ref[...] - relub
    sacc_s[...] += _dot(ohs, ca) + _dot(ohd_m, cb)

    @pl.when(i == NBLK - 1)
    def _fin():
        s_out[...] = sacc_s[...]


def _k2_body(nf_ref, ef_ref, src_ref, dst_ref, s_ref, qw_ref, poolb_ref,
             ww_ref, qf_out, wf_out, qf_s, qw2_s, qw3_s):
    i = pl.program_id(0)

    @pl.when(i == 0)
    def _init():
        relub = _relu(poolb_ref[...])
        pf = relub + s_ref[...] * (1.0 / (N - 1))
        qf = _relu(_dot(nf_ref[...], qw_ref[0:D, :]) +
                   _dot(pf, qw_ref[D:2 * D, :]))
        qf_s[...] = qf
        qf_out[...] = qf
        qw2_s[...] = _dot(qf, ww_ref[D:2 * D, :])
        qw3_s[...] = _dot(qf, ww_ref[2 * D:3 * D, :])

    src_row = src_ref[0]
    dst_row = dst_ref[0]
    iota_n = lax.broadcasted_iota(jnp.int32, (N, EBLK), 0)
    ohs = (iota_n == src_row).astype(jnp.float32)
    ohd = (iota_n == dst_row).astype(jnp.float32)
    wf_out[...] = _relu(_dot(ef_ref[...], ww_ref[0:D, :]) +
                        _dot_c0(ohs, qw2_s[...]) +
                        _dot_c0(ohd, qw3_s[...]))


def _k1a(nf, ef, src3, dst3, P_weights, pool_W, pool_b):
    return pl.pallas_call(
        _k1a_body,
        grid=(NBLK,),
        in_specs=[
            pl.BlockSpec((N, D), lambda i: (0, 0)),
            pl.BlockSpec((EBLK, D), lambda i: (i, 0)),
            pl.BlockSpec((1, 1, EBLK), lambda i: (i, 0, 0)),
            pl.BlockSpec((1, 1, EBLK), lambda i: (i, 0, 0)),
            pl.BlockSpec((3 * D, D), lambda i: (0, 0)),
            pl.BlockSpec((D, D), lambda i: (0, 0)),
            pl.BlockSpec((1, D), lambda i: (0, 0)),
        ],
        out_specs=[
            pl.BlockSpec((EBLK, D), lambda i: (i, 0)),
            pl.BlockSpec((EBLK, D), lambda i: (i, 0)),
        ],
        out_shape=[
            jax.ShapeDtypeStruct((E, D), jnp.float32),
            jax.ShapeDtypeStruct((E, D), jnp.float32),
        ],
        scratch_shapes=[
            pltpu.VMEM((N, D), jnp.float32),
        ],
    )(nf, ef, src3, dst3, P_weights, pool_W, pool_b)


def _k1b(revfeat, prea, h2b, src3, dst3, rp13, P_weights, pool_W, pool_b):
    return pl.pallas_call(
        _k1b_body,
        grid=(NBLK,),
        in_specs=[
            pl.BlockSpec((EBLK, D), lambda i: (i, 0)),
            pl.BlockSpec((EBLK, D), lambda i: (i, 0)),
            pl.BlockSpec((EBLK, D), lambda i: (i, 0)),
            pl.BlockSpec((1, 1, EBLK), lambda i: (i, 0, 0)),
            pl.BlockSpec((1, 1, EBLK), lambda i: (i, 0, 0)),
            pl.BlockSpec((1, 1, EBLK), lambda i: (i, 0, 0)),
            pl.BlockSpec((3 * D, D), lambda i: (0, 0)),
            pl.BlockSpec((D, D), lambda i: (0, 0)),
            pl.BlockSpec((1, D), lambda i: (0, 0)),
        ],
        out_specs=pl.BlockSpec((N, D), lambda i: (0, 0)),
        out_shape=jax.ShapeDtypeStruct((N, D), jnp.float32),
        scratch_shapes=[
            pltpu.VMEM((N, D), jnp.float32),
        ],
    )(revfeat, prea, h2b, src3, dst3, rp13, P_weights, pool_W, pool_b)


def _qf_wf(nf, ef, src3, dst3, s_sum, Q_weights, pool_b, W_weight):
    return pl.pallas_call(
        _k2_body,
        grid=(NBLK,),
        in_specs=[
            pl.BlockSpec((N, D), lambda i: (0, 0)),
            pl.BlockSpec((EBLK, D), lambda i: (i, 0)),
            pl.BlockSpec((1, 1, EBLK), lambda i: (i, 0, 0)),
            pl.BlockSpec((1, 1, EBLK), lambda i: (i, 0, 0)),
            pl.BlockSpec((N, D), lambda i: (0, 0)),
            pl.BlockSpec((2 * D, D), lambda i: (0, 0)),
            pl.BlockSpec((1, D), lambda i: (0, 0)),
            pl.BlockSpec((3 * D, D), lambda i: (0, 0)),
        ],
        out_specs=[
            pl.BlockSpec((N, D), lambda i: (0, 0)),
            pl.BlockSpec((EBLK, D), lambda i: (i, 0)),
        ],
        out_shape=[
            jax.ShapeDtypeStruct((N, D), jnp.float32),
            jax.ShapeDtypeStruct((E, D), jnp.float32),
        ],
        scratch_shapes=[
            pltpu.VMEM((N, D), jnp.float32),
            pltpu.VMEM((N, D), jnp.float32),
            pltpu.VMEM((N, D), jnp.float32),
        ],
    )(nf, ef, src3, dst3, s_sum, Q_weights, pool_b, W_weight)


_CHUNK = 1024


def _sc_rev_body(src_hbm, dst_hbm, ef_hbm, rp1_hbm, revfeat_hbm,
                 table_v, srcc_v, dstc_v, osrc_v, odst_v, idx_v, rows_v,
                 zrow_v, ef_sp, sem, sem_stage, sem_ping, sem_pong):
    cid = lax.axis_index("c")
    sid = lax.axis_index("s")
    wid = sid * 2 + cid

    @pl.when(sid == 0)
    def _stage():
        pltpu.async_copy(ef_hbm, ef_sp.at[pl.ds(1, E)], sem_stage)

    with jax.named_scope("sc_zero"):
        zeros16 = jnp.zeros((16,), jnp.int32)

        def zero_body(i, carry):
            for k in range(8):
                table_v[pl.ds(i * 128 + k * 16, 16)] = zeros16
            return carry

        lax.fori_loop(0, (N * N // 2) // 128, zero_body, 0)

    iota = lax.iota(jnp.int32, 16)
    nch = E // _CHUNK

    with jax.named_scope("sc_scatter"):
        halfc = _CHUNK // 2
        bufs = [(srcc_v.at[pl.ds(0, halfc)], dstc_v.at[pl.ds(0, halfc)],
                 sem_ping),
                (srcc_v.at[pl.ds(halfc, halfc)],
                 dstc_v.at[pl.ds(halfc, halfc)], sem_pong)]
        nch2 = E // halfc

        def start(c):
            sb, db, sm = bufs[c % 2]
            ds = pltpu.async_copy(src_hbm.at[pl.ds(c * halfc, halfc)], sb, sm)
            dd = pltpu.async_copy(dst_hbm.at[pl.ds(c * halfc, halfc)], db, sm)
            return ds, dd

        pend = start(0)
        for c in range(nch2):
            sb, db, _ = bufs[c % 2]
            for d in pend:
                d.wait()
            if c + 1 < nch2:
                pend = start(c + 1)

            def scatter_body(i, carry, sb=sb, db=db, c=c):
                for k in range(4):
                    off = i * 64 + k * 16
                    sv = sb[pl.ds(off, 16)]
                    dv = db[pl.ds(off, 16)]
                    linv = sv * N + dv
                    val = ((iota + (c * halfc + 1) + off)
                           << ((linv & 1) * 16))
                    plsc.addupdate_scatter(table_v, [linv >> 1], val)
                return carry

            lax.fori_loop(0, halfc // 64, scatter_body, 0)

    base = wid * EPW
    with jax.named_scope("sc_resolve"):
        pltpu.sync_copy(src_hbm.at[pl.ds(base, EPW)], osrc_v)
        pltpu.sync_copy(dst_hbm.at[pl.ds(base, EPW)], odst_v)

        def resolve_body(i, carry):
            rl = odst_v[pl.ds(i * 16, 16)] * N + osrc_v[pl.ds(i * 16, 16)]
            w = plsc.load_gather(table_v, [rl >> 1])
            rp1 = (w >> ((rl & 1) * 16)) & 0xFFFF
            idx_v[pl.ds(i * 16, 16)] = rp1
            return carry

        lax.fori_loop(0, EPW // 16, resolve_body, 0)

    @pl.when(sid == 0)
    def _stage_wait():
        zf = jnp.zeros((16,), jnp.float32)
        for k in range(8):
            zrow_v[pl.ds(k * 16, 16)] = zf
        pltpu.sync_copy(zrow_v, ef_sp.at[0])
        pltpu.make_async_copy(ef_hbm, ef_sp.at[pl.ds(1, E)],
                              sem_stage).wait()

    plsc.subcore_barrier()

    half = EPW // 2
    with jax.named_scope("sc_gather"):
        g0 = pltpu.async_copy(ef_sp.at[idx_v.at[pl.ds(0, half)]],
                              rows_v, sem)
        pltpu.sync_copy(idx_v, rp1_hbm.at[pl.ds(base, EPW)])
        g0.wait()
        pltpu.sync_copy(rows_v, revfeat_hbm.at[pl.ds(base, half)])
        g1 = pltpu.async_copy(ef_sp.at[idx_v.at[pl.ds(half, half)]],
                              rows_v, sem)
        g1.wait()
        pltpu.sync_copy(rows_v, revfeat_hbm.at[pl.ds(base + half, half)])


def _sc_rev(src, dst, ef):
    mesh = plsc.VectorSubcoreMesh(core_axis_name="c", subcore_axis_name="s")
    fn = pl.kernel(
        _sc_rev_body,
        out_type=[
            jax.ShapeDtypeStruct((E,), jnp.int32),
            jax.ShapeDtypeStruct((E, D), jnp.float32),
        ],
        mesh=mesh,
        scratch_types=[
            pltpu.VMEM((N * N // 2,), jnp.int32),
            pltpu.VMEM((_CHUNK,), jnp.int32),
            pltpu.VMEM((_CHUNK,), jnp.int32),
            pltpu.VMEM((EPW,), jnp.int32),
            pltpu.VMEM((EPW,), jnp.int32),
            pltpu.VMEM((EPW,), jnp.int32),
            pltpu.VMEM((EPW // 2, D), jnp.float32),
            pltpu.VMEM((D,), jnp.float32),
            pltpu.VMEM_SHARED((E + 1, D), jnp.float32),
            pltpu.SemaphoreType.DMA,
            pltpu.SemaphoreType.DMA,
            pltpu.SemaphoreType.DMA,
            pltpu.SemaphoreType.DMA,
        ],
        compiler_params=pltpu.CompilerParams(needs_layout_passes=False),
    )
    return fn(src, dst, ef)


def kernel(node_features, edge_features, edge_index, P_weights, Q_weights,
           pool_W, pool_b, W_weight):
    src = edge_index[:, 0]
    dst = edge_index[:, 1]
    rp1, revfeat = _sc_rev(src, dst, edge_features)

    src3 = src.reshape(NBLK, 1, EBLK)
    dst3 = dst.reshape(NBLK, 1, EBLK)
    rp13 = rp1.reshape(NBLK, 1, EBLK)
    pool_b2 = pool_b.reshape(1, D)

    prea, h2b = _k1a(node_features, edge_features, src3, dst3,
                     P_weights, pool_W, pool_b2)
    s_sum = _k1b(revfeat, prea, h2b, src3, dst3, rp13,
                 P_weights, pool_W, pool_b2)
    qf, wf = _qf_wf(node_features, edge_features, src3, dst3, s_sum,
                    Q_weights, pool_b2, W_weight)
    return (qf, wf)


# speedup vs baseline: 8.6458x; 1.0021x over previous
import functools

import jax
import jax.numpy as jnp
from jax import lax
from jax.experimental import pallas as pl
from jax.experimental.pallas import tpu as pltpu
from jax.experimental.pallas import tpu_sc as plsc

N = 256
E = 8192
D = 128
EBLK = 512
NBLK = E // EBLK
NWORK = 32
EPW = E // NWORK


def _relu(x):
    return jnp.maximum(x, 0.0)


def _dot(a, b):
    return jax.lax.dot_general(a, b, (((1,), (0,)), ((), ())),
                               preferred_element_type=jnp.float32)


def _dot_t(a, b):
    return jax.lax.dot_general(a, b, (((1,), (1,)), ((), ())),
                               preferred_element_type=jnp.float32)


def _dot_c0(a, b):
    return jax.lax.dot_general(a, b, (((0,), (0,)), ((), ())),
                               preferred_element_type=jnp.float32)


def _k1a_body(nf_ref, ef_ref, src_ref, dst_ref, pw_ref, poolw_ref,
              poolb_ref, prea_out, h2b_out, nodep1_s):
    i = pl.program_id(0)

    @pl.when(i == 0)
    def _init():
        nodep1_s[...] = _dot(nf_ref[...], pw_ref[0:D, :])

    src_row = src_ref[0]
    dst_row = dst_ref[0]
    iota_n = lax.broadcasted_iota(jnp.int32, (N, EBLK), 0)
    ohs = (iota_n == src_row).astype(jnp.bfloat16)
    ohd = (iota_n == dst_row).astype(jnp.bfloat16)

    ef_b = ef_ref[...]
    p2 = pw_ref[D:2 * D, :]
    p3 = pw_ref[2 * D:3 * D, :]
    a_f = _dot(ef_b, p2)
    b_f = _dot(ef_b, p3)
    np1_bf = nodep1_s[...].astype(jnp.bfloat16)
    n_s = _dot_c0(ohs, np1_bf)
    n_d = _dot_c0(ohd, np1_bf)

    poolb = poolb_ref[...]
    prea_out[...] = n_s + a_f
    h1b = _relu(n_d + b_f)
    h2b_out[...] = _relu(_dot_t(h1b, poolw_ref[...]) + poolb)


def _k1b_body(rvf_ref, prea_ref, h2b_ref, src_ref, dst_ref, rp1_ref,
              pw_ref, poolw_ref, poolb_ref, s_out, sacc_s):
    i = pl.program_id(0)

    @pl.when(i == 0)
    def _init():
        sacc_s[...] = jnp.zeros((N, D), jnp.float32)

    src_row = src_ref[0]
    dst_row = dst_ref[0]
    rp1_row = rp1_ref[0]
    iota_n = lax.broadcasted_iota(jnp.int32, (N, EBLK), 0)
    ohs = (iota_n == src_row).astype(jnp.bfloat16)
    ohd = (iota_n == dst_row).astype(jnp.bfloat16)
    maskb = (rp1_row == 0).astype(jnp.bfloat16)
    ohd_m = ohd * maskb

    rv_b = rvf_ref[...]
    p3 = pw_ref[2 * D:3 * D, :]
    r_f = _dot(rv_b, p3)

    poolb = poolb_ref[...]
    relub = _relu(poolb)
    h1a = _relu(prea_ref[...] + r_f)
    h2a = _relu(_dot_t(h1a, poolw_ref[...]) + poolb)
    ca = (h2a - relub).astype(jnp.bfloat16)
    cb = (h2b_ref[...] - relub).astype(jnp.bfloat16)
    sacc_s[...] += _dot(ohs, ca) + _dot(ohd_m, cb)

    @pl.when(i == NBLK - 1)
    def _fin():
        s_out[...] = sacc_s[...]


def _k2_body(nf_ref, ef_ref, src_ref, dst_ref, s_ref, qw_ref, poolb_ref,
             ww_ref, qf_out, wf_out, qf_s, qw2_s, qw3_s):
    i = pl.program_id(0)

    @pl.when(i == 0)
    def _init():
        relub = _relu(poolb_ref[...])
        pf = relub + s_ref[...] * (1.0 / (N - 1))
        qf = _relu(_dot(nf_ref[...], qw_ref[0:D, :]) +
                   _dot(pf, qw_ref[D:2 * D, :]))
        qf_s[...] = qf
        qf_out[...] = qf
        qw2_s[...] = _dot(qf, ww_ref[D:2 * D, :])
        qw3_s[...] = _dot(qf, ww_ref[2 * D:3 * D, :])

    src_row = src_ref[0]
    dst_row = dst_ref[0]
    iota_n = lax.broadcasted_iota(jnp.int32, (N, EBLK), 0)
    ohs = (iota_n == src_row).astype(jnp.bfloat16)
    ohd = (iota_n == dst_row).astype(jnp.bfloat16)
    wf_out[...] = _relu(_dot(ef_ref[...], ww_ref[0:D, :]) +
                        _dot_c0(ohs, qw2_s[...].astype(jnp.bfloat16)) +
                        _dot_c0(ohd, qw3_s[...].astype(jnp.bfloat16)))


def _k1a(nf, ef, src3, dst3, P_weights, pool_W, pool_b):
    return pl.pallas_call(
        _k1a_body,
        grid=(NBLK,),
        in_specs=[
            pl.BlockSpec((N, D), lambda i: (0, 0)),
            pl.BlockSpec((EBLK, D), lambda i: (i, 0)),
            pl.BlockSpec((1, 1, EBLK), lambda i: (i, 0, 0)),
            pl.BlockSpec((1, 1, EBLK), lambda i: (i, 0, 0)),
            pl.BlockSpec((3 * D, D), lambda i: (0, 0)),
            pl.BlockSpec((D, D), lambda i: (0, 0)),
            pl.BlockSpec((1, D), lambda i: (0, 0)),
        ],
        out_specs=[
            pl.BlockSpec((EBLK, D), lambda i: (i, 0)),
            pl.BlockSpec((EBLK, D), lambda i: (i, 0)),
        ],
        out_shape=[
            jax.ShapeDtypeStruct((E, D), jnp.float32),
            jax.ShapeDtypeStruct((E, D), jnp.float32),
        ],
        scratch_shapes=[
            pltpu.VMEM((N, D), jnp.float32),
        ],
    )(nf, ef, src3, dst3, P_weights, pool_W, pool_b)


def _k1b(revfeat, prea, h2b, src3, dst3, rp13, P_weights, pool_W, pool_b):
    return pl.pallas_call(
        _k1b_body,
        grid=(NBLK,),
        in_specs=[
            pl.BlockSpec((EBLK, D), lambda i: (i, 0)),
            pl.BlockSpec((EBLK, D), lambda i: (i, 0)),
            pl.BlockSpec((EBLK, D), lambda i: (i, 0)),
            pl.BlockSpec((1, 1, EBLK), lambda i: (i, 0, 0)),
            pl.BlockSpec((1, 1, EBLK), lambda i: (i, 0, 0)),
            pl.BlockSpec((1, 1, EBLK), lambda i: (i, 0, 0)),
            pl.BlockSpec((3 * D, D), lambda i: (0, 0)),
            pl.BlockSpec((D, D), lambda i: (0, 0)),
            pl.BlockSpec((1, D), lambda i: (0, 0)),
        ],
        out_specs=pl.BlockSpec((N, D), lambda i: (0, 0)),
        out_shape=jax.ShapeDtypeStruct((N, D), jnp.float32),
        scratch_shapes=[
            pltpu.VMEM((N, D), jnp.float32),
        ],
    )(revfeat, prea, h2b, src3, dst3, rp13, P_weights, pool_W, pool_b)


def _qf_wf(nf, ef, src3, dst3, s_sum, Q_weights, pool_b, W_weight):
    return pl.pallas_call(
        _k2_body,
        grid=(NBLK,),
        in_specs=[
            pl.BlockSpec((N, D), lambda i: (0, 0)),
            pl.BlockSpec((EBLK, D), lambda i: (i, 0)),
            pl.BlockSpec((1, 1, EBLK), lambda i: (i, 0, 0)),
            pl.BlockSpec((1, 1, EBLK), lambda i: (i, 0, 0)),
            pl.BlockSpec((N, D), lambda i: (0, 0)),
            pl.BlockSpec((2 * D, D), lambda i: (0, 0)),
            pl.BlockSpec((1, D), lambda i: (0, 0)),
            pl.BlockSpec((3 * D, D), lambda i: (0, 0)),
        ],
        out_specs=[
            pl.BlockSpec((N, D), lambda i: (0, 0)),
            pl.BlockSpec((EBLK, D), lambda i: (i, 0)),
        ],
        out_shape=[
            jax.ShapeDtypeStruct((N, D), jnp.float32),
            jax.ShapeDtypeStruct((E, D), jnp.float32),
        ],
        scratch_shapes=[
            pltpu.VMEM((N, D), jnp.float32),
            pltpu.VMEM((N, D), jnp.float32),
            pltpu.VMEM((N, D), jnp.float32),
        ],
    )(nf, ef, src3, dst3, s_sum, Q_weights, pool_b, W_weight)


_CHUNK = 1024


def _sc_rev_body(src_hbm, dst_hbm, ef_hbm, rp1_hbm, revfeat_hbm,
                 table_v, srcc_v, dstc_v, osrc_v, odst_v, idx_v, rows_v,
                 zrow_v, ef_sp, sem, sem_stage, sem_ping, sem_pong):
    cid = lax.axis_index("c")
    sid = lax.axis_index("s")
    wid = sid * 2 + cid

    @pl.when(sid == 0)
    def _stage():
        pltpu.async_copy(ef_hbm, ef_sp.at[pl.ds(1, E)], sem_stage)

    with jax.named_scope("sc_zero"):
        zeros16 = jnp.zeros((16,), jnp.int32)

        def _zero(i, carry):
            for k in range(8):
                table_v[pl.ds(i * 128 + k * 16, 16)] = zeros16
            return carry

        lax.fori_loop(0, (N * N // 2) // 128, _zero, 0)

    iota = lax.iota(jnp.int32, 16)

    with jax.named_scope("sc_scatter"):
        halfc = _CHUNK // 2
        bufs = [(srcc_v.at[pl.ds(0, halfc)], dstc_v.at[pl.ds(0, halfc)],
                 sem_ping),
                (srcc_v.at[pl.ds(halfc, halfc)],
                 dstc_v.at[pl.ds(halfc, halfc)], sem_pong)]
        nch2 = E // halfc

        def start(c):
            sb, db, sm = bufs[c % 2]
            ds = pltpu.async_copy(src_hbm.at[pl.ds(c * halfc, halfc)], sb, sm)
            dd = pltpu.async_copy(dst_hbm.at[pl.ds(c * halfc, halfc)], db, sm)
            return ds, dd

        pend = start(0)
        for c in range(nch2):
            sb, db, _ = bufs[c % 2]
            for d in pend:
                d.wait()
            if c + 1 < nch2:
                pend = start(c + 1)

            def _scat(i, carry, sb=sb, db=db, c=c):
                for k in range(4):
                    off = i * 64 + k * 16
                    sv = sb[pl.ds(off, 16)]
                    dv = db[pl.ds(off, 16)]
                    linv = sv * N + dv
                    val = ((iota + (c * halfc + 1) + off)
                           << ((linv & 1) * 16))
                    plsc.addupdate_scatter(table_v, [linv >> 1], val)
                return carry

            lax.fori_loop(0, halfc // 64, _scat, 0)

    base = wid * EPW
    with jax.named_scope("sc_resolve"):
        pltpu.sync_copy(src_hbm.at[pl.ds(base, EPW)], osrc_v)
        pltpu.sync_copy(dst_hbm.at[pl.ds(base, EPW)], odst_v)

        def _resolve(i, carry):
            rl = odst_v[pl.ds(i * 16, 16)] * N + osrc_v[pl.ds(i * 16, 16)]
            w = plsc.load_gather(table_v, [rl >> 1])
            rp1 = (w >> ((rl & 1) * 16)) & 0xFFFF
            idx_v[pl.ds(i * 16, 16)] = rp1
            return carry

        lax.fori_loop(0, EPW // 16, _resolve, 0)

    @pl.when(sid == 0)
    def _stage_wait():
        zf = jnp.zeros((16,), jnp.float32)
        for k in range(8):
            zrow_v[pl.ds(k * 16, 16)] = zf
        pltpu.sync_copy(zrow_v, ef_sp.at[0])
        pltpu.make_async_copy(ef_hbm, ef_sp.at[pl.ds(1, E)],
                              sem_stage).wait()

    plsc.subcore_barrier()

    half = EPW // 2
    with jax.named_scope("sc_gather"):
        g0 = pltpu.async_copy(ef_sp.at[idx_v.at[pl.ds(0, half)]],
                              rows_v, sem)
        pltpu.sync_copy(idx_v, rp1_hbm.at[pl.ds(base, EPW)])
        g0.wait()
        pltpu.sync_copy(rows_v, revfeat_hbm.at[pl.ds(base, half)])
        g1 = pltpu.async_copy(ef_sp.at[idx_v.at[pl.ds(half, half)]],
                              rows_v, sem)
        g1.wait()
        pltpu.sync_copy(rows_v, revfeat_hbm.at[pl.ds(base + half, half)])


def _sc_rev(src, dst, ef):
    mesh = plsc.VectorSubcoreMesh(core_axis_name="c", subcore_axis_name="s")
    fn = pl.kernel(
        _sc_rev_body,
        out_type=[
            jax.ShapeDtypeStruct((E,), jnp.int32),
            jax.ShapeDtypeStruct((E, D), jnp.float32),
        ],
        mesh=mesh,
        scratch_types=[
            pltpu.VMEM((N * N // 2,), jnp.int32),
            pltpu.VMEM((_CHUNK,), jnp.int32),
            pltpu.VMEM((_CHUNK,), jnp.int32),
            pltpu.VMEM((EPW,), jnp.int32),
            pltpu.VMEM((EPW,), jnp.int32),
            pltpu.VMEM((EPW,), jnp.int32),
            pltpu.VMEM((EPW // 2, D), jnp.float32),
            pltpu.VMEM((D,), jnp.float32),
            pltpu.VMEM_SHARED((E + 1, D), jnp.float32),
            pltpu.SemaphoreType.DMA,
            pltpu.SemaphoreType.DMA,
            pltpu.SemaphoreType.DMA,
            pltpu.SemaphoreType.DMA,
        ],
        compiler_params=pltpu.CompilerParams(needs_layout_passes=False),
    )
    return fn(src, dst, ef)


def kernel(node_features, edge_features, edge_index, P_weights, Q_weights,
           pool_W, pool_b, W_weight):
    src = edge_index[:, 0]
    dst = edge_index[:, 1]
    rp1, revfeat = _sc_rev(src, dst, edge_features)

    src3 = src.reshape(NBLK, 1, EBLK)
    dst3 = dst.reshape(NBLK, 1, EBLK)
    rp13 = rp1.reshape(NBLK, 1, EBLK)
    pool_b2 = pool_b.reshape(1, D)

    prea, h2b = _k1a(node_features, edge_features, src3, dst3,
                     P_weights, pool_W, pool_b2)
    s_sum = _k1b(revfeat, prea, h2b, src3, dst3, rp13,
                 P_weights, pool_W, pool_b2)
    qf, wf = _qf_wf(node_features, edge_features, src3, dst3, s_sum,
                    Q_weights, pool_b2, W_weight)
    return (qf, wf)


# speedup vs baseline: 10.1453x; 1.1734x over previous
import functools

import jax
import jax.numpy as jnp
from jax import lax
from jax.experimental import pallas as pl
from jax.experimental.pallas import tpu as pltpu
from jax.experimental.pallas import tpu_sc as plsc

N = 256
E = 8192
D = 128
EBLK = 1024
NBLK = E // EBLK
NWORK = 32
EPW = E // NWORK


def _relu(x):
    return jnp.maximum(x, 0.0)


def _dot(a, b):
    return jax.lax.dot_general(a, b, (((1,), (0,)), ((), ())),
                               preferred_element_type=jnp.float32)


def _dot_t(a, b):
    return jax.lax.dot_general(a, b, (((1,), (1,)), ((), ())),
                               preferred_element_type=jnp.float32)


def _dot_c0(a, b):
    return jax.lax.dot_general(a, b, (((0,), (0,)), ((), ())),
                               preferred_element_type=jnp.float32)


def _k1a_body(nf_ref, ef_ref, src_ref, dst_ref, pw_ref, poolw_ref,
              poolb_ref, prea_out, h2b_out, nodep1_s):
    i = pl.program_id(0)

    @pl.when(i == 0)
    def _init():
        nodep1_s[...] = _dot(nf_ref[...], pw_ref[0:D, :])

    src_row = src_ref[0]
    dst_row = dst_ref[0]
    iota_n = lax.broadcasted_iota(jnp.int32, (N, EBLK), 0)
    ohs = (iota_n == src_row).astype(jnp.bfloat16)
    ohd = (iota_n == dst_row).astype(jnp.bfloat16)

    ef_b = ef_ref[...]
    p2 = pw_ref[D:2 * D, :]
    p3 = pw_ref[2 * D:3 * D, :]
    a_f = _dot(ef_b, p2)
    b_f = _dot(ef_b, p3)
    np1_bf = nodep1_s[...].astype(jnp.bfloat16)
    n_s = _dot_c0(ohs, np1_bf)
    n_d = _dot_c0(ohd, np1_bf)

    poolb = poolb_ref[...]
    prea_out[...] = (n_s + a_f).astype(jnp.bfloat16)
    h1b = _relu(n_d + b_f)
    h2b_out[...] = _relu(_dot_t(h1b, poolw_ref[...]) + poolb).astype(
        jnp.bfloat16)


def _k1b_body(rvf_ref, prea_ref, h2b_ref, src_ref, dst_ref, rp1_ref,
              pw_ref, poolw_ref, poolb_ref, s_out, sacc_s):
    i = pl.program_id(0)

    @pl.when(i == 0)
    def _init():
        sacc_s[...] = jnp.zeros((N, D), jnp.float32)

    src_row = src_ref[0]
    dst_row = dst_ref[0]
    rp1_row = rp1_ref[0]
    iota_n = lax.broadcasted_iota(jnp.int32, (N, EBLK), 0)
    ohs = (iota_n == src_row).astype(jnp.bfloat16)
    ohd = (iota_n == dst_row).astype(jnp.bfloat16)
    maskb = (rp1_row == 0).astype(jnp.bfloat16)
    ohd_m = ohd * maskb

    rv_b = rvf_ref[...]
    p3 = pw_ref[2 * D:3 * D, :]
    r_f = _dot(rv_b, p3)

    poolb = poolb_ref[...]
    relub = _relu(poolb)
    h1a = _relu(prea_ref[...].astype(jnp.float32) + r_f)
    h2a = _relu(_dot_t(h1a, poolw_ref[...]) + poolb)
    ca = (h2a - relub).astype(jnp.bfloat16)
    cb = (h2b_ref[...].astype(jnp.float32) - relub).astype(jnp.bfloat16)
    sacc_s[...] += _dot(ohs, ca) + _dot(ohd_m, cb)

    @pl.when(i == NBLK - 1)
    def _fin():
        s_out[...] = sacc_s[...]


def _k2_body(nf_ref, ef_ref, src_ref, dst_ref, s_ref, qw_ref, poolb_ref,
             ww_ref, qf_out, wf_out, qf_s, qw2_s, qw3_s):
    i = pl.program_id(0)

    @pl.when(i == 0)
    def _init():
        relub = _relu(poolb_ref[...])
        pf = relub + s_ref[...] * (1.0 / (N - 1))
        qf = _relu(_dot(nf_ref[...], qw_ref[0:D, :]) +
                   _dot(pf, qw_ref[D:2 * D, :]))
        qf_s[...] = qf
        qf_out[...] = qf
        qw2_s[...] = _dot(qf, ww_ref[D:2 * D, :])
        qw3_s[...] = _dot(qf, ww_ref[2 * D:3 * D, :])

    src_row = src_ref[0]
    dst_row = dst_ref[0]
    iota_n = lax.broadcasted_iota(jnp.int32, (N, EBLK), 0)
    ohs = (iota_n == src_row).astype(jnp.bfloat16)
    ohd = (iota_n == dst_row).astype(jnp.bfloat16)
    wf_out[...] = _relu(_dot(ef_ref[...], ww_ref[0:D, :]) +
                        _dot_c0(ohs, qw2_s[...].astype(jnp.bfloat16)) +
                        _dot_c0(ohd, qw3_s[...].astype(jnp.bfloat16)))


def _k1a(nf, ef, src3, dst3, P_weights, pool_W, pool_b):
    return pl.pallas_call(
        _k1a_body,
        grid=(NBLK,),
        in_specs=[
            pl.BlockSpec((N, D), lambda i: (0, 0)),
            pl.BlockSpec((EBLK, D), lambda i: (i, 0)),
            pl.BlockSpec((1, 1, EBLK), lambda i: (i, 0, 0)),
            pl.BlockSpec((1, 1, EBLK), lambda i: (i, 0, 0)),
            pl.BlockSpec((3 * D, D), lambda i: (0, 0)),
            pl.BlockSpec((D, D), lambda i: (0, 0)),
            pl.BlockSpec((1, D), lambda i: (0, 0)),
        ],
        out_specs=[
            pl.BlockSpec((EBLK, D), lambda i: (i, 0)),
            pl.BlockSpec((EBLK, D), lambda i: (i, 0)),
        ],
        out_shape=[
            jax.ShapeDtypeStruct((E, D), jnp.bfloat16),
            jax.ShapeDtypeStruct((E, D), jnp.bfloat16),
        ],
        scratch_shapes=[
            pltpu.VMEM((N, D), jnp.float32),
        ],
    )(nf, ef, src3, dst3, P_weights, pool_W, pool_b)


def _k1b(revfeat, prea, h2b, src3, dst3, rp13, P_weights, pool_W, pool_b):
    return pl.pallas_call(
        _k1b_body,
        grid=(NBLK,),
        in_specs=[
            pl.BlockSpec((EBLK, D), lambda i: (i, 0)),
            pl.BlockSpec((EBLK, D), lambda i: (i, 0)),
            pl.BlockSpec((EBLK, D), lambda i: (i, 0)),
            pl.BlockSpec((1, 1, EBLK), lambda i: (i, 0, 0)),
            pl.BlockSpec((1, 1, EBLK), lambda i: (i, 0, 0)),
            pl.BlockSpec((1, 1, EBLK), lambda i: (i, 0, 0)),
            pl.BlockSpec((3 * D, D), lambda i: (0, 0)),
            pl.BlockSpec((D, D), lambda i: (0, 0)),
            pl.BlockSpec((1, D), lambda i: (0, 0)),
        ],
        out_specs=pl.BlockSpec((N, D), lambda i: (0, 0)),
        out_shape=jax.ShapeDtypeStruct((N, D), jnp.float32),
        scratch_shapes=[
            pltpu.VMEM((N, D), jnp.float32),
        ],
    )(revfeat, prea, h2b, src3, dst3, rp13, P_weights, pool_W, pool_b)


def _qf_wf(nf, ef, src3, dst3, s_sum, Q_weights, pool_b, W_weight):
    return pl.pallas_call(
        _k2_body,
        grid=(NBLK,),
        in_specs=[
            pl.BlockSpec((N, D), lambda i: (0, 0)),
            pl.BlockSpec((EBLK, D), lambda i: (i, 0)),
            pl.BlockSpec((1, 1, EBLK), lambda i: (i, 0, 0)),
            pl.BlockSpec((1, 1, EBLK), lambda i: (i, 0, 0)),
            pl.BlockSpec((N, D), lambda i: (0, 0)),
            pl.BlockSpec((2 * D, D), lambda i: (0, 0)),
            pl.BlockSpec((1, D), lambda i: (0, 0)),
            pl.BlockSpec((3 * D, D), lambda i: (0, 0)),
        ],
        out_specs=[
            pl.BlockSpec((N, D), lambda i: (0, 0)),
            pl.BlockSpec((EBLK, D), lambda i: (i, 0)),
        ],
        out_shape=[
            jax.ShapeDtypeStruct((N, D), jnp.float32),
            jax.ShapeDtypeStruct((E, D), jnp.float32),
        ],
        scratch_shapes=[
            pltpu.VMEM((N, D), jnp.float32),
            pltpu.VMEM((N, D), jnp.float32),
            pltpu.VMEM((N, D), jnp.float32),
        ],
    )(nf, ef, src3, dst3, s_sum, Q_weights, pool_b, W_weight)


_CHUNK = 1024


def _sc_rev_body(src_hbm, dst_hbm, ef_hbm, rp1_hbm, revfeat_hbm,
                 table_v, srcc_v, dstc_v, osrc_v, odst_v, idx_v, rows_v,
                 zrow_v, ef_sp, sem, sem_stage, sem_ping, sem_pong):
    cid = lax.axis_index("c")
    sid = lax.axis_index("s")
    wid = sid * 2 + cid

    @pl.when(sid == 0)
    def _stage():
        pltpu.async_copy(ef_hbm, ef_sp.at[pl.ds(1, E)], sem_stage)

    with jax.named_scope("sc_zero"):
        zeros16 = jnp.zeros((16,), jnp.int32)

        def _zero(i, carry):
            for k in range(8):
                table_v[pl.ds(i * 128 + k * 16, 16)] = zeros16
            return carry

        lax.fori_loop(0, (N * N // 2) // 128, _zero, 0)

    iota = lax.iota(jnp.int32, 16)

    with jax.named_scope("sc_scatter"):
        halfc = _CHUNK // 2
        bufs = [(srcc_v.at[pl.ds(0, halfc)], dstc_v.at[pl.ds(0, halfc)],
                 sem_ping),
                (srcc_v.at[pl.ds(halfc, halfc)],
                 dstc_v.at[pl.ds(halfc, halfc)], sem_pong)]
        nch2 = E // halfc

        def start(c):
            sb, db, sm = bufs[c % 2]
            ds = pltpu.async_copy(src_hbm.at[pl.ds(c * halfc, halfc)], sb, sm)
            dd = pltpu.async_copy(dst_hbm.at[pl.ds(c * halfc, halfc)], db, sm)
            return ds, dd

        pend = start(0)
        for c in range(nch2):
            sb, db, _ = bufs[c % 2]
            for d in pend:
                d.wait()
            if c + 1 < nch2:
                pend = start(c + 1)

            def _scat(i, carry, sb=sb, db=db, c=c):
                for k in range(4):
                    off = i * 64 + k * 16
                    sv = sb[pl.ds(off, 16)]
                    dv = db[pl.ds(off, 16)]
                    linv = sv * N + dv
                    val = ((iota + (c * halfc + 1) + off)
                           << ((linv & 1) * 16))
                    plsc.addupdate_scatter(table_v, [linv >> 1], val)
                return carry

            lax.fori_loop(0, halfc // 64, _scat, 0)

    base = wid * EPW
    with jax.named_scope("sc_resolve"):
        pltpu.sync_copy(src_hbm.at[pl.ds(base, EPW)], osrc_v)
        pltpu.sync_copy(dst_hbm.at[pl.ds(base, EPW)], odst_v)

        def _resolve(i, carry):
            rl = odst_v[pl.ds(i * 16, 16)] * N + osrc_v[pl.ds(i * 16, 16)]
            w = plsc.load_gather(table_v, [rl >> 1])
            rp1 = (w >> ((rl & 1) * 16)) & 0xFFFF
            idx_v[pl.ds(i * 16, 16)] = rp1
            return carry

        lax.fori_loop(0, EPW // 16, _resolve, 0)

    @pl.when(sid == 0)
    def _stage_wait():
        zf = jnp.zeros((16,), jnp.float32)
        for k in range(8):
            zrow_v[pl.ds(k * 16, 16)] = zf
        pltpu.sync_copy(zrow_v, ef_sp.at[0])
        pltpu.make_async_copy(ef_hbm, ef_sp.at[pl.ds(1, E)],
                              sem_stage).wait()

    plsc.subcore_barrier()

    half = EPW // 2
    with jax.named_scope("sc_gather"):
        g0 = pltpu.async_copy(ef_sp.at[idx_v.at[pl.ds(0, half)]],
                              rows_v, sem)
        pltpu.sync_copy(idx_v, rp1_hbm.at[pl.ds(base, EPW)])
        g0.wait()
        pltpu.sync_copy(rows_v, revfeat_hbm.at[pl.ds(base, half)])
        g1 = pltpu.async_copy(ef_sp.at[idx_v.at[pl.ds(half, half)]],
                              rows_v, sem)
        g1.wait()
        pltpu.sync_copy(rows_v, revfeat_hbm.at[pl.ds(base + half, half)])


def _sc_rev(src, dst, ef):
    mesh = plsc.VectorSubcoreMesh(core_axis_name="c", subcore_axis_name="s")
    fn = pl.kernel(
        _sc_rev_body,
        out_type=[
            jax.ShapeDtypeStruct((E,), jnp.int32),
            jax.ShapeDtypeStruct((E, D), jnp.float32),
        ],
        mesh=mesh,
        scratch_types=[
            pltpu.VMEM((N * N // 2,), jnp.int32),
            pltpu.VMEM((_CHUNK,), jnp.int32),
            pltpu.VMEM((_CHUNK,), jnp.int32),
            pltpu.VMEM((EPW,), jnp.int32),
            pltpu.VMEM((EPW,), jnp.int32),
            pltpu.VMEM((EPW,), jnp.int32),
            pltpu.VMEM((EPW // 2, D), jnp.float32),
            pltpu.VMEM((D,), jnp.float32),
            pltpu.VMEM_SHARED((E + 1, D), jnp.float32),
            pltpu.SemaphoreType.DMA,
            pltpu.SemaphoreType.DMA,
            pltpu.SemaphoreType.DMA,
            pltpu.SemaphoreType.DMA,
        ],
        compiler_params=pltpu.CompilerParams(needs_layout_passes=False),
    )
    return fn(src, dst, ef)


def kernel(node_features, edge_features, edge_index, P_weights, Q_weights,
           pool_W, pool_b, W_weight):
    src = edge_index[:, 0]
    dst = edge_index[:, 1]
    rp1, revfeat = _sc_rev(src, dst, edge_features)

    src3 = src.reshape(NBLK, 1, EBLK)
    dst3 = dst.reshape(NBLK, 1, EBLK)
    rp13 = rp1.reshape(NBLK, 1, EBLK)
    pool_b2 = pool_b.reshape(1, D)

    prea, h2b = _k1a(node_features, edge_features, src3, dst3,
                     P_weights, pool_W, pool_b2)
    s_sum = _k1b(revfeat, prea, h2b, src3, dst3, rp13,
                 P_weights, pool_W, pool_b2)
    qf, wf = _qf_wf(node_features, edge_features, src3, dst3, s_sum,
                    Q_weights, pool_b2, W_weight)
    return (qf, wf)
